# baseline (speedup 1.0000x reference)
import functools
import math

import jax
import jax.numpy as jnp
from jax import lax
from jax.experimental import pallas as pl
from jax.experimental.pallas import tpu as pltpu
from jax.experimental.pallas import tpu_sc as plsc

N = 10000
E = 160000
D0 = 256
D1 = 128
L = 2
H = 8
DH = 32
NB = 128
NG = 128
AVG_DEG = 23.395238876342773
AVG_NODES = 77.81317
MAX_R = 6.0

W = 128
NW = 80
NPAD = W * NW
B = 256
E2 = 184320
NBLK = E2 // B
BE = 2048
NBN = 512
CMSG = D0 + D1 * 3
CKVV = D0 + CMSG
CPK = 512


def _lnk(x):
    m = x.mean(-1, keepdims=True)
    var = ((x - m) ** 2).mean(-1, keepdims=True)
    return (x - m) / jnp.sqrt(var + 1e-5)


def _dot(a, b):
    return lax.dot_general(a, b, (((1,), (0,)), ((), ())),
                           preferred_element_type=jnp.float32)


NWK = 32
GCH = 64


def _sc_gather(table, idx):
    C = table.shape[1]
    dtype = table.dtype
    per_w = E2 // NWK
    nch = per_w // GCH
    mesh = plsc.VectorSubcoreMesh(core_axis_name="c", subcore_axis_name="s")

    @functools.partial(
        pl.kernel, mesh=mesh,
        out_type=jax.ShapeDtypeStruct((E2, C), dtype),
        scratch_types=[
            pltpu.VMEM((GCH,), jnp.int32),
            pltpu.VMEM((GCH,), jnp.int32),
            pltpu.VMEM((GCH, C), dtype),
            pltpu.VMEM((GCH, C), dtype),
            pltpu.SemaphoreType.DMA,
            pltpu.SemaphoreType.DMA,
            pltpu.SemaphoreType.DMA,
            pltpu.SemaphoreType.DMA,
            pltpu.SemaphoreType.DMA,
            pltpu.SemaphoreType.DMA,
        ],
    )
    def k(table_hbm, idx_hbm, out_hbm, idx0, idx1, r0, r1,
          si0, si1, sg0, sg1, so0, so1):
        wid = lax.axis_index("s") * 2 + lax.axis_index("c")
        base = wid * per_w

        @pl.loop(0, nch // 2)
        def _(t):
            j0 = 2 * t
            j1 = j0 + 1
            i0 = pltpu.async_copy(
                idx_hbm.at[pl.ds(base + j0 * GCH, GCH)], idx0, si0)
            i1 = pltpu.async_copy(
                idx_hbm.at[pl.ds(base + j1 * GCH, GCH)], idx1, si1)
            i0.wait()
            c0 = pltpu.async_copy(table_hbm.at[idx0], r0, sg0)
            i1.wait()
            c1 = pltpu.async_copy(table_hbm.at[idx1], r1, sg1)
            c0.wait()
            w0 = pltpu.async_copy(r0, out_hbm.at[pl.ds(base + j0 * GCH, GCH)],
                                  so0)
            c1.wait()
            w1 = pltpu.async_copy(r1, out_hbm.at[pl.ds(base + j1 * GCH, GCH)],
                                  so1)
            w0.wait()
            w1.wait()

    return k(table, idx)


def _onehot(dst_col, base):
    cols = lax.broadcasted_iota(jnp.int32, (B, W), 1)
    return (dst_col - base == cols)


def _spread_s():
    rows = lax.broadcasted_iota(jnp.int32, (H, D0), 0)
    cols = lax.broadcasted_iota(jnp.int32, (H, D0), 1)
    return (cols // DH == rows).astype(jnp.float32)


def _spread_v():
    rows = lax.broadcasted_iota(jnp.int32, (H, D1 * 3), 0)
    cols = lax.broadcasted_iota(jnp.int32, (H, D1 * 3), 1)
    return ((cols % D1) // (D1 // H) == rows).astype(jnp.float32)


def _headsum():
    rows = lax.broadcasted_iota(jnp.int32, (H * DH, H), 0)
    cols = lax.broadcasted_iota(jnp.int32, (H * DH, H), 1)
    return (rows // DH == cols).astype(jnp.float32)


def _edge_hh(ps, pd, Wd1, Wd2, We2):
    rel = ps[:, 0:3] - pd[:, 0:3]
    d2 = (rel * rel).sum(axis=1, keepdims=True) + 1e-12
    dist = jnp.sqrt(d2)
    inv = 1.0 / dist
    step = MAX_R / (NB - 1)
    width = MAX_R / NB
    centers = lax.broadcasted_iota(jnp.int32, (1, NB), 1).astype(jnp.float32) * step
    t = (dist - centers) * (1.0 / width)
    rbf = jnp.exp(-0.5 * t * t)
    h = rbf @ Wd1
    h = h * jax.nn.sigmoid(h)
    h = h @ Wd2
    h = h * jax.nn.sigmoid(h)
    sh = rel * inv
    hh = jnp.concatenate(
        [h, h * sh[:, 0:1], h * sh[:, 1:2], h * sh[:, 2:3]],
        axis=1).astype(jnp.bfloat16)
    return hh, rbf @ We2


def _degree_kernel(wb_ref, dst_ref, ps_ref, pd_ref, Wd1_ref, Wd2_ref, We2_ref,
                   out_ref, eb_ref, acc):
    i = pl.program_id(0)
    cur = wb_ref[i]
    prevw = wb_ref[jnp.maximum(i - 1, 0)]
    first = (i == 0) | (prevw != cur)

    @pl.when(first)
    def _():
        acc[...] = jnp.zeros_like(acc)

    hh, eb = _edge_hh(ps_ref[...], pd_ref[...], Wd1_ref[...], Wd2_ref[...],
                      We2_ref[...])
    eb_ref[...] = eb
    oh = _onehot(dst_ref[...], cur * W).astype(jnp.bfloat16)
    acc[...] += lax.dot_general(oh, hh, (((0,), (0,)), ((), ())),
                                preferred_element_type=jnp.float32)

    @pl.when(wb_ref[i + 1] != cur)
    def _():
        out_ref[...] = acc[...]


def _degree_pass(wb_ext, dst2col, pos_src, pos_dst, Wd1, Wd2, We2):
    grid_spec = pltpu.PrefetchScalarGridSpec(
        num_scalar_prefetch=1,
        grid=(NBLK,),
        in_specs=[
            pl.BlockSpec((B, 1), lambda i, wb: (i, 0)),
            pl.BlockSpec((B, 16), lambda i, wb: (i, 0)),
            pl.BlockSpec((B, 16), lambda i, wb: (i, 0)),
            pl.BlockSpec((NB, 64), lambda i, wb: (0, 0)),
            pl.BlockSpec((64, 64), lambda i, wb: (0, 0)),
            pl.BlockSpec((NB, 2 * H), lambda i, wb: (0, 0)),
        ],
        out_specs=[
            pl.BlockSpec((W, 256), lambda i, wb: (wb[i], 0)),
            pl.BlockSpec((B, 2 * H), lambda i, wb: (i, 0)),
        ],
        scratch_shapes=[pltpu.VMEM((W, 256), jnp.float32)],
    )
    return pl.pallas_call(
        _degree_kernel,
        grid_spec=grid_spec,
        out_shape=[
            jax.ShapeDtypeStruct((NPAD, 256), jnp.float32),
            jax.ShapeDtypeStruct((E2, 2 * H), jnp.float32),
        ],
    )(wb_ext, dst2col, pos_src, pos_dst, Wd1, Wd2, We2)


def _make_attn_kernel(l):
    def _attn_kernel(wb_ref, dst_ref, kvv_ref, eb_ref, q_ref, out_ref, acc, den):
        i = pl.program_id(0)
        cur = wb_ref[i]
        prevw = wb_ref[jnp.maximum(i - 1, 0)]
        first = (i == 0) | (prevw != cur)

        @pl.when(first)
        def _():
            acc[...] = jnp.zeros_like(acc)
            den[...] = jnp.zeros_like(den)

        oh = _onehot(dst_ref[...], cur * W).astype(jnp.bfloat16)
        q_e = lax.dot_general(oh, q_ref[...], (((1,), (0,)), ((), ())),
                              preferred_element_type=jnp.float32)
        val = kvv_ref[:, 0:448]
        even = lax.bitcast_convert_type(lax.shift_left(val, 16), jnp.float32)
        odd = lax.bitcast_convert_type(
            jnp.bitwise_and(val, jnp.int32(-65536)), jnp.float32)
        kvv = jnp.concatenate([even, odd], axis=1)
        k_b = kvv[:, 0:D0]
        prod = (q_e * k_b).astype(jnp.bfloat16)
        alpha = lax.dot_general(prod, _headsum().astype(jnp.bfloat16),
                                (((1,), (0,)), ((), ())),
                                preferred_element_type=jnp.float32)
        ex = jnp.exp(alpha + eb_ref[:, l * H:(l + 1) * H])
        w_s = lax.dot_general(ex, _spread_s(), (((1,), (0,)), ((), ())),
                              preferred_element_type=jnp.float32)
        w_v = lax.dot_general(ex, _spread_v(), (((1,), (0,)), ((), ())),
                              preferred_element_type=jnp.float32)
        payload = jnp.concatenate(
            [kvv[:, D0:2 * D0] * w_s,
             kvv[:, 2 * D0:CKVV] * w_v],
            axis=1).astype(jnp.bfloat16)
        acc[...] += lax.dot_general(oh, payload, (((0,), (0,)), ((), ())),
                                    preferred_element_type=jnp.float32)
        den[...] += lax.dot_general(oh, ex.astype(jnp.bfloat16),
                                    (((0,), (0,)), ((), ())),
                                    preferred_element_type=jnp.float32)

        @pl.when(wb_ref[i + 1] != cur)
        def _():
            d_s = lax.dot_general(den[...], _spread_s(), (((1,), (0,)), ((), ())),
                                  preferred_element_type=jnp.float32)
            d_v = lax.dot_general(den[...], _spread_v(), (((1,), (0,)), ((), ())),
                                  preferred_element_type=jnp.float32)
            dfull = jnp.concatenate([d_s, d_v], axis=1)
            out_ref[...] = jnp.where(dfull > 0.0, acc[...] / dfull, 0.0)

    return _attn_kernel


def _attn_pass(wb_ext, dst2col, kvv_rows, eb, q_pad, l):
    grid_spec = pltpu.PrefetchScalarGridSpec(
        num_scalar_prefetch=1,
        grid=(NBLK,),
        in_specs=[
            pl.BlockSpec((B, 1), lambda i, wb: (i, 0)),
            pl.BlockSpec((B, CPK), lambda i, wb: (i, 0)),
            pl.BlockSpec((B, 2 * H), lambda i, wb: (i, 0)),
            pl.BlockSpec((W, D0), lambda i, wb: (wb[i], 0)),
        ],
        out_specs=pl.BlockSpec((W, CMSG), lambda i, wb: (wb[i], 0)),
        scratch_shapes=[
            pltpu.VMEM((W, CMSG), jnp.float32),
            pltpu.VMEM((W, H), jnp.float32),
        ],
    )
    return pl.pallas_call(
        _make_attn_kernel(l),
        grid_spec=grid_spec,
        out_shape=jax.ShapeDtypeStruct((NPAD, CMSG), jnp.float32),
    )(wb_ext, dst2col, kvv_rows, eb, q_pad)


def _degproj_kernel(s0_ref, A_ref, Wd3s_ref, Wd3v_ref, s_ref, v_ref):
    c = 1.0 / math.sqrt(AVG_DEG)
    s_ref[...] = s0_ref[...] + _dot(A_ref[:, 0:64], Wd3s_ref[...]) * c
    for d in range(3):
        v_ref[:, d * D1:(d + 1) * D1] = _dot(
            A_ref[:, 64 + 64 * d:128 + 64 * d], Wd3v_ref[...]) * c


def _degproj(s0_pad, A, Wd3s, Wd3v):
    return pl.pallas_call(
        _degproj_kernel,
        grid=(NPAD // NBN,),
        in_specs=[
            pl.BlockSpec((NBN, D0), lambda i: (i, 0)),
            pl.BlockSpec((NBN, 256), lambda i: (i, 0)),
            pl.BlockSpec((64, D0), lambda i: (0, 0)),
            pl.BlockSpec((64, D1), lambda i: (0, 0)),
        ],
        out_specs=[
            pl.BlockSpec((NBN, D0), lambda i: (i, 0)),
            pl.BlockSpec((NBN, 3 * D1), lambda i: (i, 0)),
        ],
        out_shape=[
            jax.ShapeDtypeStruct((NPAD, D0), jnp.float32),
            jax.ShapeDtypeStruct((NPAD, 3 * D1), jnp.float32),
        ],
    )(s0_pad, A, Wd3s, Wd3v)


def _bf16_bits(x):
    r = x.astype(jnp.bfloat16).astype(jnp.float32)
    return lax.bitcast_convert_type(r, jnp.int32)


def _pre_kernel(s_ref, v_ref, wq_ref, wk_ref, wvs_ref, wvv_ref, q_ref, kvv_ref):
    s_in = _lnk(s_ref[...])
    q_ref[...] = (_dot(s_in, wq_ref[...]) * (1.0 / math.sqrt(DH))
                  ).astype(jnp.bfloat16)
    kres = _dot(s_in, wk_ref[...])
    vsres = _dot(s_in, wvs_ref[...])
    vvs = [_dot(v_ref[:, d * D1:(d + 1) * D1], wvv_ref[...]) for d in range(3)]
    full_l = jnp.concatenate([kres, vsres[:, 0:448 - D0]], axis=1)
    full_r = jnp.concatenate([vsres[:, 448 - D0:], vvs[0], vvs[1], vvs[2],
                              jnp.zeros((NBN, CPK - 448), jnp.float32)], axis=1)
    lb = lax.shift_right_logical(_bf16_bits(full_l), 16)
    rb = jnp.bitwise_and(_bf16_bits(jnp.pad(full_r, ((0, 0), (0, 0)))),
                         jnp.int32(-65536))
    kvv_ref[...] = jnp.bitwise_or(
        jnp.concatenate([lb, jnp.zeros((NBN, CPK - 448), jnp.int32)], axis=1),
        rb)


def _pre_layer(s_pad, v_pad, Wq, Wk, Wvs, Wvv):
    return pl.pallas_call(
        _pre_kernel,
        grid=(NPAD // NBN,),
        in_specs=[
            pl.BlockSpec((NBN, D0), lambda i: (i, 0)),
            pl.BlockSpec((NBN, 3 * D1), lambda i: (i, 0)),
            pl.BlockSpec((D0, D0), lambda i: (0, 0)),
            pl.BlockSpec((D0, D0), lambda i: (0, 0)),
            pl.BlockSpec((D0, D0), lambda i: (0, 0)),
            pl.BlockSpec((D1, D1), lambda i: (0, 0)),
        ],
        out_specs=[
            pl.BlockSpec((NBN, D0), lambda i: (i, 0)),
            pl.BlockSpec((NBN, CPK), lambda i: (i, 0)),
        ],
        out_shape=[
            jax.ShapeDtypeStruct((NPAD, D0), jnp.bfloat16),
            jax.ShapeDtypeStruct((NPAD, CPK), jnp.int32),
        ],
    )(s_pad, v_pad, Wq, Wk, Wvs, Wvv)


def _post_kernel(s_ref, v_ref, msg_ref, wos_ref, wov_ref, wf1_ref, wf2_ref,
                 wg0_ref, wg1_ref, wg2_ref, so_ref, vo_ref):
    s = s_ref[...] + _dot(msg_ref[:, 0:D0], wos_ref[...])
    vparts = [v_ref[:, d * D1:(d + 1) * D1] +
              _dot(msg_ref[:, D0 + d * D1:D0 + (d + 1) * D1], wov_ref[...])
              for d in range(3)]
    s_n = _lnk(s)
    hidden = _dot(s_n, wf1_ref[...])
    s = s + _dot(hidden * jax.nn.sigmoid(hidden), wf2_ref[...])
    gate = jax.nn.sigmoid(_dot(s_n, wg0_ref[...]))
    so_ref[...] = s
    for d in range(3):
        vmid = _dot(vparts[d], wg1_ref[...]) * gate
        vo_ref[:, d * D1:(d + 1) * D1] = vparts[d] + _dot(vmid, wg2_ref[...])


def _post_layer(s_pad, v_pad, msg, Wos, Wov, Wf1, Wf2, Wg0, Wg1, Wg2):
    ws = (Wos, Wov, Wf1, Wf2, Wg0, Wg1, Wg2)
    specs_w = [pl.BlockSpec(w.shape, lambda i: (0, 0)) for w in ws]
    return pl.pallas_call(
        _post_kernel,
        grid=(NPAD // NBN,),
        in_specs=[
            pl.BlockSpec((NBN, D0), lambda i: (i, 0)),
            pl.BlockSpec((NBN, 3 * D1), lambda i: (i, 0)),
            pl.BlockSpec((NBN, CMSG), lambda i: (i, 0)),
        ] + specs_w,
        out_specs=[
            pl.BlockSpec((NBN, D0), lambda i: (i, 0)),
            pl.BlockSpec((NBN, 3 * D1), lambda i: (i, 0)),
        ],
        out_shape=[
            jax.ShapeDtypeStruct((NPAD, D0), jnp.float32),
            jax.ShapeDtypeStruct((NPAD, 3 * D1), jnp.float32),
        ],
    )(s_pad, v_pad, msg, *ws)


def _head_kernel(s_ref, b_ref, wh1_ref, wh2_ref, out_ref, acc):
    i = pl.program_id(0)

    @pl.when(i == 0)
    def _():
        acc[...] = jnp.zeros_like(acc)

    sf = _lnk(s_ref[...])
    hd = _dot(sf, wh1_ref[...])
    e = _dot(hd * jax.nn.sigmoid(hd), wh2_ref[...])
    cols = lax.broadcasted_iota(jnp.int32, (NBN, NG), 1)
    oh = (b_ref[...] == cols).astype(jnp.float32)
    acc[...] += lax.dot_general(oh, e, (((0,), (0,)), ((), ())),
                                preferred_element_type=jnp.float32)

    @pl.when(i == NPAD // NBN - 1)
    def _():
        out_ref[...] = acc[...] * (1.0 / AVG_NODES)


def _head(s_pad, batch_col, Wh1, Wh2p):
    return pl.pallas_call(
        _head_kernel,
        grid=(NPAD // NBN,),
        in_specs=[
            pl.BlockSpec((NBN, D0), lambda i: (i, 0)),
            pl.BlockSpec((NBN, 1), lambda i: (i, 0)),
            pl.BlockSpec((D0, D0), lambda i: (0, 0)),
            pl.BlockSpec((D0, 8), lambda i: (0, 0)),
        ],
        out_specs=pl.BlockSpec((NG, 8), lambda i: (0, 0)),
        scratch_shapes=[pltpu.VMEM((NG, 8), jnp.float32)],
        out_shape=jax.ShapeDtypeStruct((NG, 8), jnp.float32),
    )(s_pad, batch_col, Wh1, Wh2p)


def kernel(node_atom, node_tag, pos, edge_index, batch, atom_emb, tag_emb,
           Wd1, Wd2, Wd3s, Wd3v, Wq, Wk, Wvs, Wvv, We, Wos, Wov,
           Wf1, Wf2, Wg0, Wg1, Wg2, Wh1, Wh2):
    src = edge_index[0]
    dst = edge_index[1]
    n = pos.shape[0]

    perm = jnp.argsort(dst)
    srcp = src[perm].astype(jnp.int32)
    dstp = dst[perm].astype(jnp.int32)
    p = jnp.searchsorted(dstp, jnp.arange(0, NPAD + 1, W, dtype=jnp.int32)
                         ).astype(jnp.int32)
    cnt = p[1:] - p[:-1]
    nblk_w = jnp.maximum(1, (cnt + B - 1) // B)
    pB = jnp.concatenate([jnp.zeros((1,), jnp.int32),
                          jnp.cumsum(nblk_w * B).astype(jnp.int32)])
    j = jnp.arange(E2, dtype=jnp.int32)
    wj = jnp.clip(jnp.searchsorted(pB, j, side='right') - 1, 0, NW - 1
                  ).astype(jnp.int32)
    rel_j = j - pB[wj]
    is_pad = rel_j >= cnt[wj]
    old = jnp.clip(p[wj] + rel_j, 0, E - 1)
    spread_idx = jnp.bitwise_and(j, 8191)
    dst2 = jnp.where(is_pad, -1, dstp[old])
    src2 = jnp.where(is_pad, spread_idx, srcp[old])
    dst2g = jnp.where(is_pad, spread_idx, dstp[old])
    dst2col = dst2.reshape(E2, 1)
    wb = wj[::B]
    wb_ext = jnp.concatenate([wb, jnp.full((8,), NW, jnp.int32)])

    s0 = atom_emb[node_atom] + tag_emb[node_tag]
    s0_pad = jnp.pad(s0, ((0, NPAD - n), (0, 0)))

    pos_pad = jnp.pad(pos, ((0, 0), (0, 13)))
    pos_src = pos_pad[src2]
    pos_dst = pos_pad[dst2g]
    We2 = jnp.concatenate([We[0], We[1]], axis=1)
    A, eb = _degree_pass(wb_ext, dst2col, pos_src, pos_dst, Wd1, Wd2, We2)
    s_pad, v_pad = _degproj(s0_pad, A, Wd3s, Wd3v)

    for l in range(L):
        q_pad, kvv_tab = _pre_layer(s_pad, v_pad, Wq[l], Wk[l], Wvs[l], Wvv[l])
        kvv_rows = _sc_gather(kvv_tab, src2)
        msg = _attn_pass(wb_ext, dst2col, kvv_rows, eb, q_pad, l)
        s_pad, v_pad = _post_layer(s_pad, v_pad, msg, Wos[l], Wov[l],
                                   Wf1[l], Wf2[l], Wg0[l], Wg1[l], Wg2[l])

    batch_col = jnp.pad(batch.astype(jnp.int32), (0, NPAD - n),
                        constant_values=-1).reshape(NPAD, 1)
    Wh2p = jnp.pad(Wh2, ((0, 0), (0, 7)))
    energy8 = _head(s_pad, batch_col, Wh1, Wh2p)
    return energy8[:, 0:1]


# speedup vs baseline: 4.0333x; 4.0333x over previous
import functools
import math

import jax
import jax.numpy as jnp
from jax import lax
from jax.experimental import pallas as pl
from jax.experimental.pallas import tpu as pltpu
from jax.experimental.pallas import tpu_sc as plsc

N = 10000
E = 160000
D0 = 256
D1 = 128
L = 2
H = 8
DH = 32
NB = 128
NG = 128
AVG_DEG = 23.395238876342773
AVG_NODES = 77.81317
MAX_R = 6.0

W = 128
NW = 80
NPAD = W * NW
B = 512
EPAD = 163840
BE = 2048
NBN = 512
CMSG = D0 + D1 * 3
CKVV = D0 + CMSG
CPK = 512


def _lnk(x):
    m = x.mean(-1, keepdims=True)
    var = ((x - m) ** 2).mean(-1, keepdims=True)
    return (x - m) / jnp.sqrt(var + 1e-5)


def _dot(a, b):
    return lax.dot_general(a, b, (((1,), (0,)), ((), ())),
                           preferred_element_type=jnp.float32)


def _rbf_of(ps, pd):
    rel = ps[:, 0:3] - pd[:, 0:3]
    d2 = (rel * rel).sum(axis=1, keepdims=True) + 1e-12
    dist = jnp.sqrt(d2)
    step = MAX_R / (NB - 1)
    width = MAX_R / NB
    centers = lax.broadcasted_iota(jnp.int32, (1, NB), 1).astype(jnp.float32) * step
    t = (dist - centers) * (1.0 / width)
    return jnp.exp(-0.5 * t * t), rel, 1.0 / dist


def _eb_kernel(ps_ref, pd_ref, We2_ref, eb_ref):
    rbf, _, _ = _rbf_of(ps_ref[...], pd_ref[...])
    eb_ref[...] = rbf @ We2_ref[...]


def _eb_pass(pos_src, pos_dst, We2):
    return pl.pallas_call(
        _eb_kernel,
        grid=(EPAD // BE,),
        in_specs=[
            pl.BlockSpec((BE, 16), lambda i: (i, 0)),
            pl.BlockSpec((BE, 16), lambda i: (i, 0)),
            pl.BlockSpec((NB, 2 * H), lambda i: (0, 0)),
        ],
        out_specs=pl.BlockSpec((BE, 2 * H), lambda i: (i, 0)),
        out_shape=jax.ShapeDtypeStruct((EPAD, 2 * H), jnp.float32),
    )(pos_src, pos_dst, We2)


NWK = 32
GCH = 64


def _sc_gather(table, idx):
    C = table.shape[1]
    dtype = table.dtype
    per_w = EPAD // NWK
    nch = per_w // GCH
    mesh = plsc.VectorSubcoreMesh(core_axis_name="c", subcore_axis_name="s")

    @functools.partial(
        pl.kernel, mesh=mesh,
        out_type=jax.ShapeDtypeStruct((EPAD, C), dtype),
        scratch_types=[
            pltpu.VMEM((GCH,), jnp.int32),
            pltpu.VMEM((GCH,), jnp.int32),
            pltpu.VMEM((GCH, C), dtype),
            pltpu.VMEM((GCH, C), dtype),
            pltpu.SemaphoreType.DMA,
            pltpu.SemaphoreType.DMA,
            pltpu.SemaphoreType.DMA,
            pltpu.SemaphoreType.DMA,
            pltpu.SemaphoreType.DMA,
            pltpu.SemaphoreType.DMA,
        ],
    )
    def k(table_hbm, idx_hbm, out_hbm, idx0, idx1, r0, r1,
          si0, si1, sg0, sg1, so0, so1):
        wid = lax.axis_index("s") * 2 + lax.axis_index("c")
        base = wid * per_w

        @pl.loop(0, nch // 2)
        def _(t):
            j0 = 2 * t
            j1 = j0 + 1
            i0 = pltpu.async_copy(
                idx_hbm.at[pl.ds(base + j0 * GCH, GCH)], idx0, si0)
            i1 = pltpu.async_copy(
                idx_hbm.at[pl.ds(base + j1 * GCH, GCH)], idx1, si1)
            i0.wait()
            c0 = pltpu.async_copy(table_hbm.at[idx0], r0, sg0)
            i1.wait()
            c1 = pltpu.async_copy(table_hbm.at[idx1], r1, sg1)
            c0.wait()
            w0 = pltpu.async_copy(r0, out_hbm.at[pl.ds(base + j0 * GCH, GCH)],
                                  so0)
            c1.wait()
            w1 = pltpu.async_copy(r1, out_hbm.at[pl.ds(base + j1 * GCH, GCH)],
                                  so1)
            w0.wait()
            w1.wait()

    return k(table, idx)


def _onehot(dst_col, base):
    cols = lax.broadcasted_iota(jnp.int32, (B, W), 1)
    return (dst_col - base == cols)


def _spread_s():
    rows = lax.broadcasted_iota(jnp.int32, (H, D0), 0)
    cols = lax.broadcasted_iota(jnp.int32, (H, D0), 1)
    return (cols // DH == rows).astype(jnp.float32)


def _spread_v():
    rows = lax.broadcasted_iota(jnp.int32, (H, D1 * 3), 0)
    cols = lax.broadcasted_iota(jnp.int32, (H, D1 * 3), 1)
    return ((cols % D1) // (D1 // H) == rows).astype(jnp.float32)


def _headsum():
    rows = lax.broadcasted_iota(jnp.int32, (H * DH, H), 0)
    cols = lax.broadcasted_iota(jnp.int32, (H * DH, H), 1)
    return (rows // DH == cols).astype(jnp.bfloat16)


def _degree_kernel(p_ref, dst_ref, ps_ref, pd_ref, Wd1_ref, Wd2_ref, A_ref,
                   dst_v, ps_v, pd_v, acc, sem1, sem2, sem3):
    w = pl.program_id(0)
    start = p_ref[w]
    end = p_ref[w + 1]
    base = w * W
    k0 = start // B
    nch = (end + B - 1) // B - k0
    acc[...] = jnp.zeros_like(acc)

    def body(j, _):
        off = (k0 + j) * B
        cp1 = pltpu.make_async_copy(dst_ref.at[pl.ds(off, B)], dst_v, sem1)
        cp2 = pltpu.make_async_copy(ps_ref.at[pl.ds(off, B), :], ps_v, sem2)
        cp3 = pltpu.make_async_copy(pd_ref.at[pl.ds(off, B), :], pd_v, sem3)
        cp1.start()
        cp2.start()
        cp3.start()
        cp1.wait()
        cp2.wait()
        cp3.wait()
        rbf, rel, inv = _rbf_of(ps_v[...], pd_v[...])
        h = rbf @ Wd1_ref[...]
        h = h * jax.nn.sigmoid(h)
        h = h @ Wd2_ref[...]
        h = h * jax.nn.sigmoid(h)
        sh = rel * inv
        hh = jnp.concatenate(
            [h, h * sh[:, 0:1], h * sh[:, 1:2], h * sh[:, 2:3]],
            axis=1).astype(jnp.bfloat16)
        oh = _onehot(dst_v[...].reshape(B, 1), base).astype(jnp.bfloat16)
        acc[...] += lax.dot_general(oh, hh, (((0,), (0,)), ((), ())),
                                    preferred_element_type=jnp.float32)
        return 0

    lax.fori_loop(0, nch, body, 0)
    A_ref[...] = acc[...]


def _degree_pass(p, dstp, pos_src, pos_dst, Wd1, Wd2):
    grid_spec = pltpu.PrefetchScalarGridSpec(
        num_scalar_prefetch=1,
        grid=(NW,),
        in_specs=[
            pl.BlockSpec(memory_space=pltpu.MemorySpace.HBM),
            pl.BlockSpec(memory_space=pltpu.MemorySpace.HBM),
            pl.BlockSpec(memory_space=pltpu.MemorySpace.HBM),
            pl.BlockSpec((NB, 64), lambda w, p: (0, 0)),
            pl.BlockSpec((64, 64), lambda w, p: (0, 0)),
        ],
        out_specs=pl.BlockSpec((W, 256), lambda w, p: (w, 0)),
        scratch_shapes=[
            pltpu.VMEM((B,), jnp.int32),
            pltpu.VMEM((B, 16), jnp.float32),
            pltpu.VMEM((B, 16), jnp.float32),
            pltpu.VMEM((W, 256), jnp.float32),
            pltpu.SemaphoreType.DMA,
            pltpu.SemaphoreType.DMA,
            pltpu.SemaphoreType.DMA,
        ],
    )
    return pl.pallas_call(
        _degree_kernel,
        grid_spec=grid_spec,
        out_shape=jax.ShapeDtypeStruct((NPAD, 256), jnp.float32),
    )(p, dstp, pos_src, pos_dst, Wd1, Wd2)


def _make_attn_kernel(l):
    def _attn_kernel(p_ref, dst_ref, kvv_ref, eb_ref, q_ref, out_ref,
                     dst_v, kvv_v, eb_v, acc, den, sem1, sem2, sem3):
        w = pl.program_id(0)
        start = p_ref[w]
        end = p_ref[w + 1]
        base = w * W
        k0 = start // B
        nch = (end + B - 1) // B - k0
        acc[...] = jnp.zeros_like(acc)
        den[...] = jnp.zeros_like(den)

        def body(j, _):
            off = (k0 + j) * B
            cp1 = pltpu.make_async_copy(dst_ref.at[pl.ds(off, B)], dst_v, sem1)
            cp2 = pltpu.make_async_copy(kvv_ref.at[pl.ds(off, B), :], kvv_v, sem2)
            cp3 = pltpu.make_async_copy(eb_ref.at[pl.ds(off, B), :], eb_v, sem3)
            cp1.start()
            cp2.start()
            cp3.start()
            cp1.wait()
            cp2.wait()
            cp3.wait()
            val = kvv_v[:, 0:448]
            even = lax.bitcast_convert_type(lax.shift_left(val, 16), jnp.float32)
            odd = lax.bitcast_convert_type(
                jnp.bitwise_and(val, jnp.int32(-65536)), jnp.float32)
            kvv = jnp.concatenate([even, odd], axis=1)
            oh = _onehot(dst_v[...].reshape(B, 1), base).astype(jnp.bfloat16)
            q_e = lax.dot_general(oh, q_ref[...], (((1,), (0,)), ((), ())),
                                  preferred_element_type=jnp.float32)
            prod = (q_e * kvv[:, 0:D0]).astype(jnp.bfloat16)
            alpha = lax.dot_general(prod, _headsum(), (((1,), (0,)), ((), ())),
                                    preferred_element_type=jnp.float32)
            ex = jnp.exp(alpha + eb_v[:, l * H:(l + 1) * H])
            w_s = lax.dot_general(ex, _spread_s(), (((1,), (0,)), ((), ())),
                                  preferred_element_type=jnp.float32)
            w_v = lax.dot_general(ex, _spread_v(), (((1,), (0,)), ((), ())),
                                  preferred_element_type=jnp.float32)
            payload = jnp.concatenate(
                [kvv[:, D0:2 * D0] * w_s, kvv[:, 2 * D0:CKVV] * w_v],
                axis=1).astype(jnp.bfloat16)
            acc[...] += lax.dot_general(oh, payload, (((0,), (0,)), ((), ())),
                                        preferred_element_type=jnp.float32)
            den[...] += lax.dot_general(oh, ex.astype(jnp.bfloat16),
                                        (((0,), (0,)), ((), ())),
                                        preferred_element_type=jnp.float32)
            return 0

        lax.fori_loop(0, nch, body, 0)
        d_s = lax.dot_general(den[...], _spread_s(), (((1,), (0,)), ((), ())),
                              preferred_element_type=jnp.float32)
        d_v = lax.dot_general(den[...], _spread_v(), (((1,), (0,)), ((), ())),
                              preferred_element_type=jnp.float32)
        dfull = jnp.concatenate([d_s, d_v], axis=1)
        out_ref[...] = jnp.where(dfull > 0.0, acc[...] / dfull, 0.0)

    return _attn_kernel


def _attn_pass(p, dstp, kvv_rows, eb, q_pad, l):
    grid_spec = pltpu.PrefetchScalarGridSpec(
        num_scalar_prefetch=1,
        grid=(NW,),
        in_specs=[
            pl.BlockSpec(memory_space=pltpu.MemorySpace.HBM),
            pl.BlockSpec(memory_space=pltpu.MemorySpace.HBM),
            pl.BlockSpec(memory_space=pltpu.MemorySpace.HBM),
            pl.BlockSpec((W, D0), lambda w, p: (w, 0)),
        ],
        out_specs=pl.BlockSpec((W, CMSG), lambda w, p: (w, 0)),
        scratch_shapes=[
            pltpu.VMEM((B,), jnp.int32),
            pltpu.VMEM((B, CPK), jnp.int32),
            pltpu.VMEM((B, 2 * H), jnp.float32),
            pltpu.VMEM((W, CMSG), jnp.float32),
            pltpu.VMEM((W, H), jnp.float32),
            pltpu.SemaphoreType.DMA,
            pltpu.SemaphoreType.DMA,
            pltpu.SemaphoreType.DMA,
        ],
    )
    return pl.pallas_call(
        _make_attn_kernel(l),
        grid_spec=grid_spec,
        out_shape=jax.ShapeDtypeStruct((NPAD, CMSG), jnp.float32),
    )(p, dstp, kvv_rows, eb, q_pad)


def _degproj_kernel(s0_ref, A_ref, Wd3s_ref, Wd3v_ref, s_ref, v_ref):
    c = 1.0 / math.sqrt(AVG_DEG)
    s_ref[...] = s0_ref[...] + _dot(A_ref[:, 0:64], Wd3s_ref[...]) * c
    for d in range(3):
        v_ref[:, d * D1:(d + 1) * D1] = _dot(
            A_ref[:, 64 + 64 * d:128 + 64 * d], Wd3v_ref[...]) * c


def _degproj(s0_pad, A, Wd3s, Wd3v):
    return pl.pallas_call(
        _degproj_kernel,
        grid=(NPAD // NBN,),
        in_specs=[
            pl.BlockSpec((NBN, D0), lambda i: (i, 0)),
            pl.BlockSpec((NBN, 256), lambda i: (i, 0)),
            pl.BlockSpec((64, D0), lambda i: (0, 0)),
            pl.BlockSpec((64, D1), lambda i: (0, 0)),
        ],
        out_specs=[
            pl.BlockSpec((NBN, D0), lambda i: (i, 0)),
            pl.BlockSpec((NBN, 3 * D1), lambda i: (i, 0)),
        ],
        out_shape=[
            jax.ShapeDtypeStruct((NPAD, D0), jnp.float32),
            jax.ShapeDtypeStruct((NPAD, 3 * D1), jnp.float32),
        ],
    )(s0_pad, A, Wd3s, Wd3v)


def _bf16_bits(x):
    r = x.astype(jnp.bfloat16).astype(jnp.float32)
    return lax.bitcast_convert_type(r, jnp.int32)


def _pre_kernel(s_ref, v_ref, wq_ref, wk_ref, wvs_ref, wvv_ref, q_ref, kvv_ref):
    s_in = _lnk(s_ref[...])
    q_ref[...] = (_dot(s_in, wq_ref[...]) * (1.0 / math.sqrt(DH))
                  ).astype(jnp.bfloat16)
    kres = _dot(s_in, wk_ref[...])
    vsres = _dot(s_in, wvs_ref[...])
    vvs = [_dot(v_ref[:, d * D1:(d + 1) * D1], wvv_ref[...]) for d in range(3)]
    full_l = jnp.concatenate([kres, vsres[:, 0:448 - D0]], axis=1)
    full_r = jnp.concatenate([vsres[:, 448 - D0:], vvs[0], vvs[1], vvs[2],
                              jnp.zeros((NBN, CPK - 448), jnp.float32)], axis=1)
    lb = lax.shift_right_logical(_bf16_bits(full_l), 16)
    rb = jnp.bitwise_and(_bf16_bits(full_r), jnp.int32(-65536))
    kvv_ref[...] = jnp.bitwise_or(
        jnp.concatenate([lb, jnp.zeros((NBN, CPK - 448), jnp.int32)], axis=1),
        rb)


def _pre_layer(s_pad, v_pad, Wq, Wk, Wvs, Wvv):
    return pl.pallas_call(
        _pre_kernel,
        grid=(NPAD // NBN,),
        in_specs=[
            pl.BlockSpec((NBN, D0), lambda i: (i, 0)),
            pl.BlockSpec((NBN, 3 * D1), lambda i: (i, 0)),
            pl.BlockSpec((D0, D0), lambda i: (0, 0)),
            pl.BlockSpec((D0, D0), lambda i: (0, 0)),
            pl.BlockSpec((D0, D0), lambda i: (0, 0)),
            pl.BlockSpec((D1, D1), lambda i: (0, 0)),
        ],
        out_specs=[
            pl.BlockSpec((NBN, D0), lambda i: (i, 0)),
            pl.BlockSpec((NBN, CPK), lambda i: (i, 0)),
        ],
        out_shape=[
            jax.ShapeDtypeStruct((NPAD, D0), jnp.bfloat16),
            jax.ShapeDtypeStruct((NPAD, CPK), jnp.int32),
        ],
    )(s_pad, v_pad, Wq, Wk, Wvs, Wvv)


def _post_kernel(s_ref, v_ref, msg_ref, wos_ref, wov_ref, wf1_ref, wf2_ref,
                 wg0_ref, wg1_ref, wg2_ref, so_ref, vo_ref):
    s = s_ref[...] + _dot(msg_ref[:, 0:D0], wos_ref[...])
    vparts = [v_ref[:, d * D1:(d + 1) * D1] +
              _dot(msg_ref[:, D0 + d * D1:D0 + (d + 1) * D1], wov_ref[...])
              for d in range(3)]
    s_n = _lnk(s)
    hidden = _dot(s_n, wf1_ref[...])
    s = s + _dot(hidden * jax.nn.sigmoid(hidden), wf2_ref[...])
    gate = jax.nn.sigmoid(_dot(s_n, wg0_ref[...]))
    so_ref[...] = s
    for d in range(3):
        vmid = _dot(vparts[d], wg1_ref[...]) * gate
        vo_ref[:, d * D1:(d + 1) * D1] = vparts[d] + _dot(vmid, wg2_ref[...])


def _post_layer(s_pad, v_pad, msg, Wos, Wov, Wf1, Wf2, Wg0, Wg1, Wg2):
    ws = (Wos, Wov, Wf1, Wf2, Wg0, Wg1, Wg2)
    specs_w = [pl.BlockSpec(w.shape, lambda i: (0, 0)) for w in ws]
    return pl.pallas_call(
        _post_kernel,
        grid=(NPAD // NBN,),
        in_specs=[
            pl.BlockSpec((NBN, D0), lambda i: (i, 0)),
            pl.BlockSpec((NBN, 3 * D1), lambda i: (i, 0)),
            pl.BlockSpec((NBN, CMSG), lambda i: (i, 0)),
        ] + specs_w,
        out_specs=[
            pl.BlockSpec((NBN, D0), lambda i: (i, 0)),
            pl.BlockSpec((NBN, 3 * D1), lambda i: (i, 0)),
        ],
        out_shape=[
            jax.ShapeDtypeStruct((NPAD, D0), jnp.float32),
            jax.ShapeDtypeStruct((NPAD, 3 * D1), jnp.float32),
        ],
    )(s_pad, v_pad, msg, *ws)


def _head_kernel(s_ref, b_ref, wh1_ref, wh2_ref, out_ref, acc):
    i = pl.program_id(0)

    @pl.when(i == 0)
    def _():
        acc[...] = jnp.zeros_like(acc)

    sf = _lnk(s_ref[...])
    hd = _dot(sf, wh1_ref[...])
    e = _dot(hd * jax.nn.sigmoid(hd), wh2_ref[...])
    cols = lax.broadcasted_iota(jnp.int32, (NBN, NG), 1)
    oh = (b_ref[...] == cols).astype(jnp.float32)
    acc[...] += lax.dot_general(oh, e, (((0,), (0,)), ((), ())),
                                preferred_element_type=jnp.float32)

    @pl.when(i == NPAD // NBN - 1)
    def _():
        out_ref[...] = acc[...] * (1.0 / AVG_NODES)


def _head(s_pad, batch_col, Wh1, Wh2p):
    return pl.pallas_call(
        _head_kernel,
        grid=(NPAD // NBN,),
        in_specs=[
            pl.BlockSpec((NBN, D0), lambda i: (i, 0)),
            pl.BlockSpec((NBN, 1), lambda i: (i, 0)),
            pl.BlockSpec((D0, D0), lambda i: (0, 0)),
            pl.BlockSpec((D0, 8), lambda i: (0, 0)),
        ],
        out_specs=pl.BlockSpec((NG, 8), lambda i: (0, 0)),
        scratch_shapes=[pltpu.VMEM((NG, 8), jnp.float32)],
        out_shape=jax.ShapeDtypeStruct((NG, 8), jnp.float32),
    )(s_pad, batch_col, Wh1, Wh2p)


def kernel(node_atom, node_tag, pos, edge_index, batch, atom_emb, tag_emb,
           Wd1, Wd2, Wd3s, Wd3v, Wq, Wk, Wvs, Wvv, We, Wos, Wov,
           Wf1, Wf2, Wg0, Wg1, Wg2, Wh1, Wh2):
    src = edge_index[0]
    dst = edge_index[1]
    n = pos.shape[0]

    perm = jnp.argsort(dst)
    srcp = src[perm].astype(jnp.int32)
    dstp = dst[perm].astype(jnp.int32)
    spread_idx = jnp.bitwise_and(jnp.arange(EPAD - E, dtype=jnp.int32), 8191)
    dstp_pad = jnp.concatenate(
        [dstp, jnp.full((EPAD - E,), NPAD - 1, jnp.int32)])
    srcp_pad = jnp.concatenate([srcp, spread_idx])
    dstp_clip = jnp.concatenate([dstp, spread_idx])
    p = jnp.searchsorted(dstp, jnp.arange(0, NPAD + 1, W, dtype=jnp.int32)
                         ).astype(jnp.int32)
    p = p.at[-1].set(E)

    s0 = atom_emb[node_atom] + tag_emb[node_tag]
    s0_pad = jnp.pad(s0, ((0, NPAD - n), (0, 0)))

    pos_pad = jnp.pad(pos, ((0, 0), (0, 13)))
    pos_src = pos_pad[srcp_pad]
    pos_dst = pos_pad[dstp_clip]
    We2 = jnp.concatenate([We[0], We[1]], axis=1)
    eb = _eb_pass(pos_src, pos_dst, We2)
    A = _degree_pass(p, dstp_pad, pos_src, pos_dst, Wd1, Wd2)
    s_pad, v_pad = _degproj(s0_pad, A, Wd3s, Wd3v)

    for l in range(L):
        q_pad, kvv_tab = _pre_layer(s_pad, v_pad, Wq[l], Wk[l], Wvs[l], Wvv[l])
        kvv_rows = _sc_gather(kvv_tab, srcp_pad)
        msg = _attn_pass(p, dstp_pad, kvv_rows, eb, q_pad, l)
        s_pad, v_pad = _post_layer(s_pad, v_pad, msg, Wos[l], Wov[l],
                                   Wf1[l], Wf2[l], Wg0[l], Wg1[l], Wg2[l])

    batch_col = jnp.pad(batch.astype(jnp.int32), (0, NPAD - n),
                        constant_values=-1).reshape(NPAD, 1)
    Wh2p = jnp.pad(Wh2, ((0, 0), (0, 7)))
    energy8 = _head(s_pad, batch_col, Wh1, Wh2p)
    return energy8[:, 0:1]


# speedup vs baseline: 5.1145x; 1.2681x over previous
import functools
import math

import jax
import jax.numpy as jnp
from jax import lax
from jax.experimental import pallas as pl
from jax.experimental.pallas import tpu as pltpu
from jax.experimental.pallas import tpu_sc as plsc

N = 10000
E = 160000
D0 = 256
D1 = 128
L = 2
H = 8
DH = 32
NB = 128
NG = 128
AVG_DEG = 23.395238876342773
AVG_NODES = 77.81317
MAX_R = 6.0

W = 128
NW = 80
NPAD = W * NW
B = 512
EPAD = 163840
BE = 2048
NBN = 512
CMSG = D0 + D1 * 3
CKVV = D0 + CMSG
CPK = 512


def _lnk(x):
    m = x.mean(-1, keepdims=True)
    var = ((x - m) ** 2).mean(-1, keepdims=True)
    return (x - m) / jnp.sqrt(var + 1e-5)


def _dot(a, b):
    return lax.dot_general(a, b, (((1,), (0,)), ((), ())),
                           preferred_element_type=jnp.float32)


def _rbf_of(ps, pd):
    rel = ps[:, 0:3] - pd[:, 0:3]
    d2 = (rel * rel).sum(axis=1, keepdims=True) + 1e-12
    dist = jnp.sqrt(d2)
    step = MAX_R / (NB - 1)
    width = MAX_R / NB
    centers = lax.broadcasted_iota(jnp.int32, (1, NB), 1).astype(jnp.float32) * step
    t = (dist - centers) * (1.0 / width)
    return jnp.exp(-0.5 * t * t), rel, 1.0 / dist


def _eb_kernel(ps_ref, pd_ref, We2_ref, eb_ref):
    rbf, _, _ = _rbf_of(ps_ref[...], pd_ref[...])
    eb_ref[...] = rbf @ We2_ref[...]


def _eb_pass(pos_src, pos_dst, We2):
    return pl.pallas_call(
        _eb_kernel,
        grid=(EPAD // BE,),
        in_specs=[
            pl.BlockSpec((BE, 16), lambda i: (i, 0)),
            pl.BlockSpec((BE, 16), lambda i: (i, 0)),
            pl.BlockSpec((NB, 2 * H), lambda i: (0, 0)),
        ],
        out_specs=pl.BlockSpec((BE, 2 * H), lambda i: (i, 0)),
        out_shape=jax.ShapeDtypeStruct((EPAD, 2 * H), jnp.float32),
    )(pos_src, pos_dst, We2)


NWK = 32
GCH = 80


def _sc_gather(table, idx):
    C = table.shape[1]
    dtype = table.dtype
    per_w = EPAD // NWK
    nch = per_w // GCH
    mesh = plsc.VectorSubcoreMesh(core_axis_name="c", subcore_axis_name="s")

    @functools.partial(
        pl.kernel, mesh=mesh,
        out_type=jax.ShapeDtypeStruct((EPAD, C), dtype),
        scratch_types=[
            pltpu.VMEM((GCH,), jnp.int32),
            pltpu.VMEM((GCH,), jnp.int32),
            pltpu.VMEM((GCH, C), dtype),
            pltpu.VMEM((GCH, C), dtype),
            pltpu.SemaphoreType.DMA,
            pltpu.SemaphoreType.DMA,
            pltpu.SemaphoreType.DMA,
            pltpu.SemaphoreType.DMA,
            pltpu.SemaphoreType.DMA,
            pltpu.SemaphoreType.DMA,
        ],
    )
    def k(table_hbm, idx_hbm, out_hbm, idx0, idx1, r0, r1,
          si0, si1, sg0, sg1, so0, so1):
        wid = lax.axis_index("s") * 2 + lax.axis_index("c")
        base = wid * per_w

        @pl.loop(0, nch // 2)
        def _(t):
            j0 = 2 * t
            j1 = j0 + 1
            i0 = pltpu.async_copy(
                idx_hbm.at[pl.ds(base + j0 * GCH, GCH)], idx0, si0)
            i1 = pltpu.async_copy(
                idx_hbm.at[pl.ds(base + j1 * GCH, GCH)], idx1, si1)
            i0.wait()
            c0 = pltpu.async_copy(table_hbm.at[idx0], r0, sg0)
            i1.wait()
            c1 = pltpu.async_copy(table_hbm.at[idx1], r1, sg1)
            c0.wait()
            w0 = pltpu.async_copy(r0, out_hbm.at[pl.ds(base + j0 * GCH, GCH)],
                                  so0)
            c1.wait()
            w1 = pltpu.async_copy(r1, out_hbm.at[pl.ds(base + j1 * GCH, GCH)],
                                  so1)
            w0.wait()
            w1.wait()

    return k(table, idx)


def _onehot(dst_col, base):
    cols = lax.broadcasted_iota(jnp.int32, (B, W), 1)
    return (dst_col - base == cols)


def _spread_s():
    rows = lax.broadcasted_iota(jnp.int32, (H, D0), 0)
    cols = lax.broadcasted_iota(jnp.int32, (H, D0), 1)
    return (cols // DH == rows).astype(jnp.float32)


def _spread_v():
    rows = lax.broadcasted_iota(jnp.int32, (H, D1 * 3), 0)
    cols = lax.broadcasted_iota(jnp.int32, (H, D1 * 3), 1)
    return ((cols % D1) // (D1 // H) == rows).astype(jnp.float32)


def _headsum():
    rows = lax.broadcasted_iota(jnp.int32, (H * DH, H), 0)
    cols = lax.broadcasted_iota(jnp.int32, (H * DH, H), 1)
    return (rows // DH == cols).astype(jnp.bfloat16)


def _degree_kernel(p_ref, dst_ref, ps_ref, pd_ref, Wd1_ref, Wd2_ref, A_ref,
                   dst_v0, ps_v0, pd_v0, dst_v1, ps_v1, pd_v1, acc,
                   s10, s20, s30, s11, s21, s31):
    w = pl.program_id(0)
    start = p_ref[w]
    end = p_ref[w + 1]
    base = w * W
    k0 = start // B
    nch = (end + B - 1) // B - k0
    acc[...] = jnp.zeros_like(acc)

    bufs = ((dst_v0, ps_v0, pd_v0, s10, s20, s30),
            (dst_v1, ps_v1, pd_v1, s11, s21, s31))

    def start_chunk(b, j):
        dv, pv, qv, sa, sb, sc = bufs[b]
        off = (k0 + j) * B
        pltpu.make_async_copy(dst_ref.at[pl.ds(off, B)], dv, sa).start()
        pltpu.make_async_copy(ps_ref.at[pl.ds(off, B), :], pv, sb).start()
        pltpu.make_async_copy(pd_ref.at[pl.ds(off, B), :], qv, sc).start()

    def compute_chunk(b, j):
        dv, pv, qv, sa, sb, sc = bufs[b]
        off = (k0 + j) * B
        pltpu.make_async_copy(dst_ref.at[pl.ds(off, B)], dv, sa).wait()
        pltpu.make_async_copy(ps_ref.at[pl.ds(off, B), :], pv, sb).wait()
        pltpu.make_async_copy(pd_ref.at[pl.ds(off, B), :], qv, sc).wait()
        rbf, rel, inv = _rbf_of(pv[...], qv[...])
        h = rbf @ Wd1_ref[...]
        h = h * jax.nn.sigmoid(h)
        h = h @ Wd2_ref[...]
        h = h * jax.nn.sigmoid(h)
        sh = rel * inv
        hh = jnp.concatenate(
            [h, h * sh[:, 0:1], h * sh[:, 1:2], h * sh[:, 2:3]],
            axis=1).astype(jnp.bfloat16)
        oh = _onehot(dv[...].reshape(B, 1), base).astype(jnp.bfloat16)
        acc[...] += lax.dot_general(oh, hh, (((0,), (0,)), ((), ())),
                                    preferred_element_type=jnp.float32)

    @pl.when(nch > 0)
    def _():
        start_chunk(0, 0)

    def body2(t, _):
        j0 = 2 * t
        j1 = j0 + 1

        @pl.when(j1 < nch)
        def _():
            start_chunk(1, j1)

        compute_chunk(0, j0)

        @pl.when(j1 + 1 < nch)
        def _():
            start_chunk(0, j1 + 1)

        @pl.when(j1 < nch)
        def _():
            compute_chunk(1, j1)

        return 0

    lax.fori_loop(0, (nch + 1) // 2, body2, 0)
    A_ref[...] = acc[...]


def _degree_pass(p, dstp, pos_src, pos_dst, Wd1, Wd2):
    grid_spec = pltpu.PrefetchScalarGridSpec(
        num_scalar_prefetch=1,
        grid=(NW,),
        in_specs=[
            pl.BlockSpec(memory_space=pltpu.MemorySpace.HBM),
            pl.BlockSpec(memory_space=pltpu.MemorySpace.HBM),
            pl.BlockSpec(memory_space=pltpu.MemorySpace.HBM),
            pl.BlockSpec((NB, 64), lambda w, p: (0, 0)),
            pl.BlockSpec((64, 64), lambda w, p: (0, 0)),
        ],
        out_specs=pl.BlockSpec((W, 256), lambda w, p: (w, 0)),
        scratch_shapes=[
            pltpu.VMEM((B,), jnp.int32),
            pltpu.VMEM((B, 16), jnp.float32),
            pltpu.VMEM((B, 16), jnp.float32),
            pltpu.VMEM((B,), jnp.int32),
            pltpu.VMEM((B, 16), jnp.float32),
            pltpu.VMEM((B, 16), jnp.float32),
            pltpu.VMEM((W, 256), jnp.float32),
            pltpu.SemaphoreType.DMA,
            pltpu.SemaphoreType.DMA,
            pltpu.SemaphoreType.DMA,
            pltpu.SemaphoreType.DMA,
            pltpu.SemaphoreType.DMA,
            pltpu.SemaphoreType.DMA,
        ],
    )
    return pl.pallas_call(
        _degree_kernel,
        grid_spec=grid_spec,
        out_shape=jax.ShapeDtypeStruct((NPAD, 256), jnp.float32),
    )(p, dstp, pos_src, pos_dst, Wd1, Wd2)


def _make_attn_kernel(l):
    def _attn_kernel(p_ref, dst_ref, kvv_ref, eb_ref, q_ref, out_ref,
                     dst_v0, kvv_v0, eb_v0, dst_v1, kvv_v1, eb_v1,
                     acc, den, s10, s20, s30, s11, s21, s31):
        w = pl.program_id(0)
        start = p_ref[w]
        end = p_ref[w + 1]
        base = w * W
        k0 = start // B
        nch = (end + B - 1) // B - k0
        acc[...] = jnp.zeros_like(acc)
        den[...] = jnp.zeros_like(den)

        bufs = ((dst_v0, kvv_v0, eb_v0, s10, s20, s30),
                (dst_v1, kvv_v1, eb_v1, s11, s21, s31))

        def start_chunk(b, j):
            dv, kv, ev, sa, sb, sc = bufs[b]
            off = (k0 + j) * B
            pltpu.make_async_copy(dst_ref.at[pl.ds(off, B)], dv, sa).start()
            pltpu.make_async_copy(kvv_ref.at[pl.ds(off, B), :], kv, sb).start()
            pltpu.make_async_copy(eb_ref.at[pl.ds(off, B), :], ev, sc).start()

        def compute_chunk(b, j):
            dv, kv, ev, sa, sb, sc = bufs[b]
            off = (k0 + j) * B
            pltpu.make_async_copy(dst_ref.at[pl.ds(off, B)], dv, sa).wait()
            pltpu.make_async_copy(kvv_ref.at[pl.ds(off, B), :], kv, sb).wait()
            pltpu.make_async_copy(eb_ref.at[pl.ds(off, B), :], ev, sc).wait()
            val = kv[:, 0:448]
            even = lax.bitcast_convert_type(lax.shift_left(val, 16), jnp.float32)
            odd = lax.bitcast_convert_type(
                jnp.bitwise_and(val, jnp.int32(-65536)), jnp.float32)
            kvv = jnp.concatenate([even, odd], axis=1)
            oh = _onehot(dv[...].reshape(B, 1), base).astype(jnp.bfloat16)
            q_e = lax.dot_general(oh, q_ref[...], (((1,), (0,)), ((), ())),
                                  preferred_element_type=jnp.float32)
            prod = (q_e * kvv[:, 0:D0]).astype(jnp.bfloat16)
            alpha = lax.dot_general(prod, _headsum(), (((1,), (0,)), ((), ())),
                                    preferred_element_type=jnp.float32)
            ex = jnp.exp(alpha + ev[:, l * H:(l + 1) * H])
            w_s = lax.dot_general(ex, _spread_s(), (((1,), (0,)), ((), ())),
                                  preferred_element_type=jnp.float32)
            w_v = lax.dot_general(ex, _spread_v(), (((1,), (0,)), ((), ())),
                                  preferred_element_type=jnp.float32)
            payload = jnp.concatenate(
                [kvv[:, D0:2 * D0] * w_s, kvv[:, 2 * D0:CKVV] * w_v],
                axis=1).astype(jnp.bfloat16)
            acc[...] += lax.dot_general(oh, payload, (((0,), (0,)), ((), ())),
                                        preferred_element_type=jnp.float32)
            den[...] += lax.dot_general(oh, ex.astype(jnp.bfloat16),
                                        (((0,), (0,)), ((), ())),
                                        preferred_element_type=jnp.float32)

        @pl.when(nch > 0)
        def _():
            start_chunk(0, 0)

        def body2(t, _):
            j0 = 2 * t
            j1 = j0 + 1

            @pl.when(j1 < nch)
            def _():
                start_chunk(1, j1)

            compute_chunk(0, j0)

            @pl.when(j1 + 1 < nch)
            def _():
                start_chunk(0, j1 + 1)

            @pl.when(j1 < nch)
            def _():
                compute_chunk(1, j1)

            return 0

        lax.fori_loop(0, (nch + 1) // 2, body2, 0)
        d_s = lax.dot_general(den[...], _spread_s(), (((1,), (0,)), ((), ())),
                              preferred_element_type=jnp.float32)
        d_v = lax.dot_general(den[...], _spread_v(), (((1,), (0,)), ((), ())),
                              preferred_element_type=jnp.float32)
        dfull = jnp.concatenate([d_s, d_v], axis=1)
        out_ref[...] = jnp.where(dfull > 0.0, acc[...] / dfull, 0.0)

    return _attn_kernel


def _attn_pass(p, dstp, kvv_rows, eb, q_pad, l):
    grid_spec = pltpu.PrefetchScalarGridSpec(
        num_scalar_prefetch=1,
        grid=(NW,),
        in_specs=[
            pl.BlockSpec(memory_space=pltpu.MemorySpace.HBM),
            pl.BlockSpec(memory_space=pltpu.MemorySpace.HBM),
            pl.BlockSpec(memory_space=pltpu.MemorySpace.HBM),
            pl.BlockSpec((W, D0), lambda w, p: (w, 0)),
        ],
        out_specs=pl.BlockSpec((W, CMSG), lambda w, p: (w, 0)),
        scratch_shapes=[
            pltpu.VMEM((B,), jnp.int32),
            pltpu.VMEM((B, CPK), jnp.int32),
            pltpu.VMEM((B, 2 * H), jnp.float32),
            pltpu.VMEM((B,), jnp.int32),
            pltpu.VMEM((B, CPK), jnp.int32),
            pltpu.VMEM((B, 2 * H), jnp.float32),
            pltpu.VMEM((W, CMSG), jnp.float32),
            pltpu.VMEM((W, H), jnp.float32),
            pltpu.SemaphoreType.DMA,
            pltpu.SemaphoreType.DMA,
            pltpu.SemaphoreType.DMA,
            pltpu.SemaphoreType.DMA,
            pltpu.SemaphoreType.DMA,
            pltpu.SemaphoreType.DMA,
        ],
    )
    return pl.pallas_call(
        _make_attn_kernel(l),
        grid_spec=grid_spec,
        out_shape=jax.ShapeDtypeStruct((NPAD, CMSG), jnp.float32),
    )(p, dstp, kvv_rows, eb, q_pad)


def _degproj_kernel(s0_ref, A_ref, Wd3s_ref, Wd3v_ref, s_ref, v_ref):
    c = 1.0 / math.sqrt(AVG_DEG)
    s_ref[...] = s0_ref[...] + _dot(A_ref[:, 0:64], Wd3s_ref[...]) * c
    for d in range(3):
        v_ref[:, d * D1:(d + 1) * D1] = _dot(
            A_ref[:, 64 + 64 * d:128 + 64 * d], Wd3v_ref[...]) * c


def _degproj(s0_pad, A, Wd3s, Wd3v):
    return pl.pallas_call(
        _degproj_kernel,
        grid=(NPAD // NBN,),
        in_specs=[
            pl.BlockSpec((NBN, D0), lambda i: (i, 0)),
            pl.BlockSpec((NBN, 256), lambda i: (i, 0)),
            pl.BlockSpec((64, D0), lambda i: (0, 0)),
            pl.BlockSpec((64, D1), lambda i: (0, 0)),
        ],
        out_specs=[
            pl.BlockSpec((NBN, D0), lambda i: (i, 0)),
            pl.BlockSpec((NBN, 3 * D1), lambda i: (i, 0)),
        ],
        out_shape=[
            jax.ShapeDtypeStruct((NPAD, D0), jnp.float32),
            jax.ShapeDtypeStruct((NPAD, 3 * D1), jnp.float32),
        ],
    )(s0_pad, A, Wd3s, Wd3v)


def _bf16_bits(x):
    r = x.astype(jnp.bfloat16).astype(jnp.float32)
    return lax.bitcast_convert_type(r, jnp.int32)


def _pre_kernel(s_ref, v_ref, wq_ref, wk_ref, wvs_ref, wvv_ref, q_ref, kvv_ref):
    s_in = _lnk(s_ref[...])
    q_ref[...] = (_dot(s_in, wq_ref[...]) * (1.0 / math.sqrt(DH))
                  ).astype(jnp.bfloat16)
    kres = _dot(s_in, wk_ref[...])
    vsres = _dot(s_in, wvs_ref[...])
    vvs = [_dot(v_ref[:, d * D1:(d + 1) * D1], wvv_ref[...]) for d in range(3)]
    full_l = jnp.concatenate([kres, vsres[:, 0:448 - D0]], axis=1)
    full_r = jnp.concatenate([vsres[:, 448 - D0:], vvs[0], vvs[1], vvs[2],
                              jnp.zeros((NBN, CPK - 448), jnp.float32)], axis=1)
    lb = lax.shift_right_logical(_bf16_bits(full_l), 16)
    rb = jnp.bitwise_and(_bf16_bits(full_r), jnp.int32(-65536))
    kvv_ref[...] = jnp.bitwise_or(
        jnp.concatenate([lb, jnp.zeros((NBN, CPK - 448), jnp.int32)], axis=1),
        rb)


def _pre_layer(s_pad, v_pad, Wq, Wk, Wvs, Wvv):
    return pl.pallas_call(
        _pre_kernel,
        grid=(NPAD // NBN,),
        in_specs=[
            pl.BlockSpec((NBN, D0), lambda i: (i, 0)),
            pl.BlockSpec((NBN, 3 * D1), lambda i: (i, 0)),
            pl.BlockSpec((D0, D0), lambda i: (0, 0)),
            pl.BlockSpec((D0, D0), lambda i: (0, 0)),
            pl.BlockSpec((D0, D0), lambda i: (0, 0)),
            pl.BlockSpec((D1, D1), lambda i: (0, 0)),
        ],
        out_specs=[
            pl.BlockSpec((NBN, D0), lambda i: (i, 0)),
            pl.BlockSpec((NBN, CPK), lambda i: (i, 0)),
        ],
        out_shape=[
            jax.ShapeDtypeStruct((NPAD, D0), jnp.bfloat16),
            jax.ShapeDtypeStruct((NPAD, CPK), jnp.int32),
        ],
    )(s_pad, v_pad, Wq, Wk, Wvs, Wvv)


def _post_kernel(s_ref, v_ref, msg_ref, wos_ref, wov_ref, wf1_ref, wf2_ref,
                 wg0_ref, wg1_ref, wg2_ref, so_ref, vo_ref):
    s = s_ref[...] + _dot(msg_ref[:, 0:D0], wos_ref[...])
    vparts = [v_ref[:, d * D1:(d + 1) * D1] +
              _dot(msg_ref[:, D0 + d * D1:D0 + (d + 1) * D1], wov_ref[...])
              for d in range(3)]
    s_n = _lnk(s)
    hidden = _dot(s_n, wf1_ref[...])
    s = s + _dot(hidden * jax.nn.sigmoid(hidden), wf2_ref[...])
    gate = jax.nn.sigmoid(_dot(s_n, wg0_ref[...]))
    so_ref[...] = s
    for d in range(3):
        vmid = _dot(vparts[d], wg1_ref[...]) * gate
        vo_ref[:, d * D1:(d + 1) * D1] = vparts[d] + _dot(vmid, wg2_ref[...])


def _post_layer(s_pad, v_pad, msg, Wos, Wov, Wf1, Wf2, Wg0, Wg1, Wg2):
    ws = (Wos, Wov, Wf1, Wf2, Wg0, Wg1, Wg2)
    specs_w = [pl.BlockSpec(w.shape, lambda i: (0, 0)) for w in ws]
    return pl.pallas_call(
        _post_kernel,
        grid=(NPAD // NBN,),
        in_specs=[
            pl.BlockSpec((NBN, D0), lambda i: (i, 0)),
            pl.BlockSpec((NBN, 3 * D1), lambda i: (i, 0)),
            pl.BlockSpec((NBN, CMSG), lambda i: (i, 0)),
        ] + specs_w,
        out_specs=[
            pl.BlockSpec((NBN, D0), lambda i: (i, 0)),
            pl.BlockSpec((NBN, 3 * D1), lambda i: (i, 0)),
        ],
        out_shape=[
            jax.ShapeDtypeStruct((NPAD, D0), jnp.float32),
            jax.ShapeDtypeStruct((NPAD, 3 * D1), jnp.float32),
        ],
    )(s_pad, v_pad, msg, *ws)


def _head_kernel(s_ref, b_ref, wh1_ref, wh2_ref, out_ref, acc):
    i = pl.program_id(0)

    @pl.when(i == 0)
    def _():
        acc[...] = jnp.zeros_like(acc)

    sf = _lnk(s_ref[...])
    hd = _dot(sf, wh1_ref[...])
    e = _dot(hd * jax.nn.sigmoid(hd), wh2_ref[...])
    cols = lax.broadcasted_iota(jnp.int32, (NBN, NG), 1)
    oh = (b_ref[...] == cols).astype(jnp.float32)
    acc[...] += lax.dot_general(oh, e, (((0,), (0,)), ((), ())),
                                preferred_element_type=jnp.float32)

    @pl.when(i == NPAD // NBN - 1)
    def _():
        out_ref[...] = acc[...] * (1.0 / AVG_NODES)


def _head(s_pad, batch_col, Wh1, Wh2p):
    return pl.pallas_call(
        _head_kernel,
        grid=(NPAD // NBN,),
        in_specs=[
            pl.BlockSpec((NBN, D0), lambda i: (i, 0)),
            pl.BlockSpec((NBN, 1), lambda i: (i, 0)),
            pl.BlockSpec((D0, D0), lambda i: (0, 0)),
            pl.BlockSpec((D0, 8), lambda i: (0, 0)),
        ],
        out_specs=pl.BlockSpec((NG, 8), lambda i: (0, 0)),
        scratch_shapes=[pltpu.VMEM((NG, 8), jnp.float32)],
        out_shape=jax.ShapeDtypeStruct((NG, 8), jnp.float32),
    )(s_pad, batch_col, Wh1, Wh2p)


def kernel(node_atom, node_tag, pos, edge_index, batch, atom_emb, tag_emb,
           Wd1, Wd2, Wd3s, Wd3v, Wq, Wk, Wvs, Wvv, We, Wos, Wov,
           Wf1, Wf2, Wg0, Wg1, Wg2, Wh1, Wh2):
    src = edge_index[0]
    dst = edge_index[1]
    n = pos.shape[0]

    perm = jnp.argsort(dst)
    srcp = src[perm].astype(jnp.int32)
    dstp = dst[perm].astype(jnp.int32)
    spread_idx = jnp.bitwise_and(jnp.arange(EPAD - E, dtype=jnp.int32), 8191)
    dstp_pad = jnp.concatenate(
        [dstp, jnp.full((EPAD - E,), NPAD - 1, jnp.int32)])
    srcp_pad = jnp.concatenate([srcp, spread_idx])
    dstp_clip = jnp.concatenate([dstp, spread_idx])
    p = jnp.searchsorted(dstp, jnp.arange(0, NPAD + 1, W, dtype=jnp.int32)
                         ).astype(jnp.int32)
    p = p.at[-1].set(E)

    s0 = atom_emb[node_atom] + tag_emb[node_tag]
    s0_pad = jnp.pad(s0, ((0, NPAD - n), (0, 0)))

    pos_pad = jnp.pad(pos, ((0, 0), (0, 13)))
    pos_src = pos_pad[srcp_pad]
    pos_dst = pos_pad[dstp_clip]
    We2 = jnp.concatenate([We[0], We[1]], axis=1)
    eb = _eb_pass(pos_src, pos_dst, We2)
    A = _degree_pass(p, dstp_pad, pos_src, pos_dst, Wd1, Wd2)
    s_pad, v_pad = _degproj(s0_pad, A, Wd3s, Wd3v)

    for l in range(L):
        q_pad, kvv_tab = _pre_layer(s_pad, v_pad, Wq[l], Wk[l], Wvs[l], Wvv[l])
        kvv_rows = _sc_gather(kvv_tab, srcp_pad)
        msg = _attn_pass(p, dstp_pad, kvv_rows, eb, q_pad, l)
        s_pad, v_pad = _post_layer(s_pad, v_pad, msg, Wos[l], Wov[l],
                                   Wf1[l], Wf2[l], Wg0[l], Wg1[l], Wg2[l])

    batch_col = jnp.pad(batch.astype(jnp.int32), (0, NPAD - n),
                        constant_values=-1).reshape(NPAD, 1)
    Wh2p = jnp.pad(Wh2, ((0, 0), (0, 7)))
    energy8 = _head(s_pad, batch_col, Wh1, Wh2p)
    return energy8[:, 0:1]


# speedup vs baseline: 5.2616x; 1.0288x over previous
import functools
import math

import jax
import jax.numpy as jnp
from jax import lax
from jax.experimental import pallas as pl
from jax.experimental.pallas import tpu as pltpu
from jax.experimental.pallas import tpu_sc as plsc

N = 10000
E = 160000
D0 = 256
D1 = 128
L = 2
H = 8
DH = 32
NB = 128
NG = 128
AVG_DEG = 23.395238876342773
AVG_NODES = 77.81317
MAX_R = 6.0

W = 128
NW = 80
NPAD = W * NW
B = 512
BA = 1024
EPAD = 163840
BE = 2048
NBN = 512
CMSG = D0 + D1 * 3
CKVV = D0 + CMSG
CPK = 512


def _lnk(x):
    m = x.mean(-1, keepdims=True)
    var = ((x - m) ** 2).mean(-1, keepdims=True)
    return (x - m) / jnp.sqrt(var + 1e-5)


def _dot(a, b):
    return lax.dot_general(a, b, (((1,), (0,)), ((), ())),
                           preferred_element_type=jnp.float32)


def _rbf_of(ps, pd):
    rel = ps[:, 0:3] - pd[:, 0:3]
    d2 = (rel * rel).sum(axis=1, keepdims=True) + 1e-12
    dist = jnp.sqrt(d2)
    step = MAX_R / (NB - 1)
    width = MAX_R / NB
    centers = lax.broadcasted_iota(jnp.int32, (1, NB), 1).astype(jnp.float32) * step
    t = (dist - centers) * (1.0 / width)
    return jnp.exp(-0.5 * t * t), rel, 1.0 / dist


def _eb_kernel(ps_ref, pd_ref, We2_ref, eb_ref):
    rbf, _, _ = _rbf_of(ps_ref[...], pd_ref[...])
    eb_ref[...] = rbf @ We2_ref[...]


def _eb_pass(pos_src, pos_dst, We2):
    return pl.pallas_call(
        _eb_kernel,
        grid=(EPAD // BE,),
        in_specs=[
            pl.BlockSpec((BE, 16), lambda i: (i, 0)),
            pl.BlockSpec((BE, 16), lambda i: (i, 0)),
            pl.BlockSpec((NB, 2 * H), lambda i: (0, 0)),
        ],
        out_specs=pl.BlockSpec((BE, 2 * H), lambda i: (i, 0)),
        out_shape=jax.ShapeDtypeStruct((EPAD, 2 * H), jnp.float32),
    )(pos_src, pos_dst, We2)


NWK = 32
GCH = 80


def _sc_gather(table, idx):
    C = table.shape[1]
    dtype = table.dtype
    per_w = EPAD // NWK
    nch = per_w // GCH
    mesh = plsc.VectorSubcoreMesh(core_axis_name="c", subcore_axis_name="s")

    @functools.partial(
        pl.kernel, mesh=mesh,
        out_type=jax.ShapeDtypeStruct((EPAD, C), dtype),
        scratch_types=[
            pltpu.VMEM((GCH,), jnp.int32),
            pltpu.VMEM((GCH,), jnp.int32),
            pltpu.VMEM((GCH, C), dtype),
            pltpu.VMEM((GCH, C), dtype),
            pltpu.SemaphoreType.DMA,
            pltpu.SemaphoreType.DMA,
            pltpu.SemaphoreType.DMA,
            pltpu.SemaphoreType.DMA,
            pltpu.SemaphoreType.DMA,
            pltpu.SemaphoreType.DMA,
        ],
    )
    def k(table_hbm, idx_hbm, out_hbm, idx0, idx1, r0, r1,
          si0, si1, sg0, sg1, so0, so1):
        wid = lax.axis_index("s") * 2 + lax.axis_index("c")
        base = wid * per_w

        @pl.loop(0, nch // 2)
        def _(t):
            j0 = 2 * t
            j1 = j0 + 1
            i0 = pltpu.async_copy(
                idx_hbm.at[pl.ds(base + j0 * GCH, GCH)], idx0, si0)
            i1 = pltpu.async_copy(
                idx_hbm.at[pl.ds(base + j1 * GCH, GCH)], idx1, si1)
            i0.wait()
            c0 = pltpu.async_copy(table_hbm.at[idx0], r0, sg0)
            i1.wait()
            c1 = pltpu.async_copy(table_hbm.at[idx1], r1, sg1)
            c0.wait()
            w0 = pltpu.async_copy(r0, out_hbm.at[pl.ds(base + j0 * GCH, GCH)],
                                  so0)
            c1.wait()
            w1 = pltpu.async_copy(r1, out_hbm.at[pl.ds(base + j1 * GCH, GCH)],
                                  so1)
            w0.wait()
            w1.wait()

    return k(table, idx)


def _onehot(dst_col, base, nb=None):
    cols = lax.broadcasted_iota(jnp.int32, (nb or B, W), 1)
    return (dst_col - base == cols)


def _spread_s():
    rows = lax.broadcasted_iota(jnp.int32, (H, D0), 0)
    cols = lax.broadcasted_iota(jnp.int32, (H, D0), 1)
    return (cols // DH == rows).astype(jnp.float32)


def _spread_v():
    rows = lax.broadcasted_iota(jnp.int32, (H, D1 * 3), 0)
    cols = lax.broadcasted_iota(jnp.int32, (H, D1 * 3), 1)
    return ((cols % D1) // (D1 // H) == rows).astype(jnp.float32)


def _spread_sv():
    return jnp.concatenate([_spread_s(), _spread_v()], axis=1)


def _headsum():
    rows = lax.broadcasted_iota(jnp.int32, (H * DH, H), 0)
    cols = lax.broadcasted_iota(jnp.int32, (H * DH, H), 1)
    return (rows // DH == cols).astype(jnp.bfloat16)


def _degree_kernel(p_ref, dst_ref, ps_ref, pd_ref, Wd1_ref, Wd2_ref, A_ref,
                   dst_v0, ps_v0, pd_v0, dst_v1, ps_v1, pd_v1, acc,
                   s10, s20, s30, s11, s21, s31):
    w = pl.program_id(0)
    start = p_ref[w]
    end = p_ref[w + 1]
    base = w * W
    k0 = start // B
    nch = (end + B - 1) // B - k0
    acc[...] = jnp.zeros_like(acc)

    bufs = ((dst_v0, ps_v0, pd_v0, s10, s20, s30),
            (dst_v1, ps_v1, pd_v1, s11, s21, s31))

    def start_chunk(b, j):
        dv, pv, qv, sa, sb, sc = bufs[b]
        off = (k0 + j) * B
        pltpu.make_async_copy(dst_ref.at[pl.ds(off, B)], dv, sa).start()
        pltpu.make_async_copy(ps_ref.at[pl.ds(off, B), :], pv, sb).start()
        pltpu.make_async_copy(pd_ref.at[pl.ds(off, B), :], qv, sc).start()

    def compute_chunk(b, j):
        dv, pv, qv, sa, sb, sc = bufs[b]
        off = (k0 + j) * B
        pltpu.make_async_copy(dst_ref.at[pl.ds(off, B)], dv, sa).wait()
        pltpu.make_async_copy(ps_ref.at[pl.ds(off, B), :], pv, sb).wait()
        pltpu.make_async_copy(pd_ref.at[pl.ds(off, B), :], qv, sc).wait()
        rbf, rel, inv = _rbf_of(pv[...], qv[...])
        h = rbf @ Wd1_ref[...]
        h = h * jax.nn.sigmoid(h)
        h = h @ Wd2_ref[...]
        h = h * jax.nn.sigmoid(h)
        sh = rel * inv
        hh = jnp.concatenate(
            [h, h * sh[:, 0:1], h * sh[:, 1:2], h * sh[:, 2:3]],
            axis=1).astype(jnp.bfloat16)
        oh = _onehot(dv[...].reshape(B, 1), base).astype(jnp.bfloat16)
        acc[...] += lax.dot_general(oh, hh, (((0,), (0,)), ((), ())),
                                    preferred_element_type=jnp.float32)

    @pl.when(nch > 0)
    def _():
        start_chunk(0, 0)

    def body2(t, _):
        j0 = 2 * t
        j1 = j0 + 1

        @pl.when(j1 < nch)
        def _():
            start_chunk(1, j1)

        compute_chunk(0, j0)

        @pl.when(j1 + 1 < nch)
        def _():
            start_chunk(0, j1 + 1)

        @pl.when(j1 < nch)
        def _():
            compute_chunk(1, j1)

        return 0

    lax.fori_loop(0, (nch + 1) // 2, body2, 0)
    A_ref[...] = acc[...]


def _degree_pass(p, dstp, pos_src, pos_dst, Wd1, Wd2):
    grid_spec = pltpu.PrefetchScalarGridSpec(
        num_scalar_prefetch=1,
        grid=(NW,),
        in_specs=[
            pl.BlockSpec(memory_space=pltpu.MemorySpace.HBM),
            pl.BlockSpec(memory_space=pltpu.MemorySpace.HBM),
            pl.BlockSpec(memory_space=pltpu.MemorySpace.HBM),
            pl.BlockSpec((NB, 64), lambda w, p: (0, 0)),
            pl.BlockSpec((64, 64), lambda w, p: (0, 0)),
        ],
        out_specs=pl.BlockSpec((W, 256), lambda w, p: (w, 0)),
        scratch_shapes=[
            pltpu.VMEM((B,), jnp.int32),
            pltpu.VMEM((B, 16), jnp.float32),
            pltpu.VMEM((B, 16), jnp.float32),
            pltpu.VMEM((B,), jnp.int32),
            pltpu.VMEM((B, 16), jnp.float32),
            pltpu.VMEM((B, 16), jnp.float32),
            pltpu.VMEM((W, 256), jnp.float32),
            pltpu.SemaphoreType.DMA,
            pltpu.SemaphoreType.DMA,
            pltpu.SemaphoreType.DMA,
            pltpu.SemaphoreType.DMA,
            pltpu.SemaphoreType.DMA,
            pltpu.SemaphoreType.DMA,
        ],
    )
    return pl.pallas_call(
        _degree_kernel,
        grid_spec=grid_spec,
        out_shape=jax.ShapeDtypeStruct((NPAD, 256), jnp.float32),
    )(p, dstp, pos_src, pos_dst, Wd1, Wd2)


def _make_attn_kernel(l):
    def _attn_kernel(p_ref, dst_ref, kvv_ref, eb_ref, q_ref, out_ref,
                     dst_v0, kvv_v0, eb_v0, dst_v1, kvv_v1, eb_v1,
                     acc, s10, s20, s30, s11, s21, s31):
        w = pl.program_id(0)
        start = p_ref[w]
        end = p_ref[w + 1]
        base = w * W
        k0 = start // BA
        nch = (end + BA - 1) // BA - k0
        acc[...] = jnp.zeros_like(acc)

        bufs = ((dst_v0, kvv_v0, eb_v0, s10, s20, s30),
                (dst_v1, kvv_v1, eb_v1, s11, s21, s31))

        def start_chunk(b, j):
            dv, kv, ev, sa, sb, sc = bufs[b]
            off = (k0 + j) * BA
            pltpu.make_async_copy(dst_ref.at[pl.ds(off, BA)], dv, sa).start()
            pltpu.make_async_copy(kvv_ref.at[pl.ds(off, BA), :], kv, sb).start()
            pltpu.make_async_copy(eb_ref.at[pl.ds(off, BA), :], ev, sc).start()

        def compute_chunk(b, j):
            dv, kv, ev, sa, sb, sc = bufs[b]
            off = (k0 + j) * BA
            pltpu.make_async_copy(dst_ref.at[pl.ds(off, BA)], dv, sa).wait()
            pltpu.make_async_copy(kvv_ref.at[pl.ds(off, BA), :], kv, sb).wait()
            pltpu.make_async_copy(eb_ref.at[pl.ds(off, BA), :], ev, sc).wait()
            val = kv[:, 0:448]
            even = lax.bitcast_convert_type(lax.shift_left(val, 16), jnp.float32)
            odd = lax.bitcast_convert_type(
                jnp.bitwise_and(val, jnp.int32(-65536)), jnp.float32)
            kvv = jnp.concatenate([even, odd], axis=1)
            oh = _onehot(dv[...].reshape(BA, 1), base, BA).astype(jnp.bfloat16)
            q_e = lax.dot_general(oh, q_ref[...], (((1,), (0,)), ((), ())),
                                  preferred_element_type=jnp.float32)
            prod = (q_e * kvv[:, 0:D0]).astype(jnp.bfloat16)
            alpha = lax.dot_general(prod, _headsum(), (((1,), (0,)), ((), ())),
                                    preferred_element_type=jnp.float32)
            ex = jnp.exp(alpha + ev[:, l * H:(l + 1) * H])
            w_sv = lax.dot_general(ex, _spread_sv(), (((1,), (0,)), ((), ())),
                                   preferred_element_type=jnp.float32)
            payload = jnp.concatenate(
                [kvv[:, D0:CKVV] * w_sv, ex], axis=1).astype(jnp.bfloat16)
            acc[...] += lax.dot_general(oh, payload, (((0,), (0,)), ((), ())),
                                        preferred_element_type=jnp.float32)

        @pl.when(nch > 0)
        def _():
            start_chunk(0, 0)

        def body2(t, _):
            j0 = 2 * t
            j1 = j0 + 1

            @pl.when(j1 < nch)
            def _():
                start_chunk(1, j1)

            compute_chunk(0, j0)

            @pl.when(j1 + 1 < nch)
            def _():
                start_chunk(0, j1 + 1)

            @pl.when(j1 < nch)
            def _():
                compute_chunk(1, j1)

            return 0

        lax.fori_loop(0, (nch + 1) // 2, body2, 0)
        dfull = lax.dot_general(acc[:, CMSG:CMSG + H], _spread_sv(),
                                (((1,), (0,)), ((), ())),
                                preferred_element_type=jnp.float32)
        out_ref[...] = jnp.where(dfull > 0.0, acc[:, 0:CMSG] / dfull, 0.0)

    return _attn_kernel


def _attn_pass(p, dstp, kvv_rows, eb, q_pad, l):
    grid_spec = pltpu.PrefetchScalarGridSpec(
        num_scalar_prefetch=1,
        grid=(NW,),
        in_specs=[
            pl.BlockSpec(memory_space=pltpu.MemorySpace.HBM),
            pl.BlockSpec(memory_space=pltpu.MemorySpace.HBM),
            pl.BlockSpec(memory_space=pltpu.MemorySpace.HBM),
            pl.BlockSpec((W, D0), lambda w, p: (w, 0)),
        ],
        out_specs=pl.BlockSpec((W, CMSG), lambda w, p: (w, 0)),
        scratch_shapes=[
            pltpu.VMEM((BA,), jnp.int32),
            pltpu.VMEM((BA, CPK), jnp.int32),
            pltpu.VMEM((BA, 2 * H), jnp.float32),
            pltpu.VMEM((BA,), jnp.int32),
            pltpu.VMEM((BA, CPK), jnp.int32),
            pltpu.VMEM((BA, 2 * H), jnp.float32),
            pltpu.VMEM((W, CMSG + H), jnp.float32),
            pltpu.SemaphoreType.DMA,
            pltpu.SemaphoreType.DMA,
            pltpu.SemaphoreType.DMA,
            pltpu.SemaphoreType.DMA,
            pltpu.SemaphoreType.DMA,
            pltpu.SemaphoreType.DMA,
        ],
    )
    return pl.pallas_call(
        _make_attn_kernel(l),
        grid_spec=grid_spec,
        out_shape=jax.ShapeDtypeStruct((NPAD, CMSG), jnp.float32),
    )(p, dstp, kvv_rows, eb, q_pad)


def _degproj_kernel(s0_ref, A_ref, Wd3s_ref, Wd3v_ref, s_ref, v_ref):
    c = 1.0 / math.sqrt(AVG_DEG)
    s_ref[...] = s0_ref[...] + _dot(A_ref[:, 0:64], Wd3s_ref[...]) * c
    for d in range(3):
        v_ref[:, d * D1:(d + 1) * D1] = _dot(
            A_ref[:, 64 + 64 * d:128 + 64 * d], Wd3v_ref[...]) * c


def _degproj(s0_pad, A, Wd3s, Wd3v):
    return pl.pallas_call(
        _degproj_kernel,
        grid=(NPAD // NBN,),
        in_specs=[
            pl.BlockSpec((NBN, D0), lambda i: (i, 0)),
            pl.BlockSpec((NBN, 256), lambda i: (i, 0)),
            pl.BlockSpec((64, D0), lambda i: (0, 0)),
            pl.BlockSpec((64, D1), lambda i: (0, 0)),
        ],
        out_specs=[
            pl.BlockSpec((NBN, D0), lambda i: (i, 0)),
            pl.BlockSpec((NBN, 3 * D1), lambda i: (i, 0)),
        ],
        out_shape=[
            jax.ShapeDtypeStruct((NPAD, D0), jnp.float32),
            jax.ShapeDtypeStruct((NPAD, 3 * D1), jnp.float32),
        ],
    )(s0_pad, A, Wd3s, Wd3v)


def _bf16_bits(x):
    r = x.astype(jnp.bfloat16).astype(jnp.float32)
    return lax.bitcast_convert_type(r, jnp.int32)


def _pre_kernel(s_ref, v_ref, wq_ref, wk_ref, wvs_ref, wvv_ref, q_ref, kvv_ref):
    s_in = _lnk(s_ref[...])
    q_ref[...] = (_dot(s_in, wq_ref[...]) * (1.0 / math.sqrt(DH))
                  ).astype(jnp.bfloat16)
    kres = _dot(s_in, wk_ref[...])
    vsres = _dot(s_in, wvs_ref[...])
    vvs = [_dot(v_ref[:, d * D1:(d + 1) * D1], wvv_ref[...]) for d in range(3)]
    full_l = jnp.concatenate([kres, vsres[:, 0:448 - D0]], axis=1)
    full_r = jnp.concatenate([vsres[:, 448 - D0:], vvs[0], vvs[1], vvs[2],
                              jnp.zeros((NBN, CPK - 448), jnp.float32)], axis=1)
    lb = lax.shift_right_logical(_bf16_bits(full_l), 16)
    rb = jnp.bitwise_and(_bf16_bits(full_r), jnp.int32(-65536))
    kvv_ref[...] = jnp.bitwise_or(
        jnp.concatenate([lb, jnp.zeros((NBN, CPK - 448), jnp.int32)], axis=1),
        rb)


def _pre_layer(s_pad, v_pad, Wq, Wk, Wvs, Wvv):
    return pl.pallas_call(
        _pre_kernel,
        grid=(NPAD // NBN,),
        in_specs=[
            pl.BlockSpec((NBN, D0), lambda i: (i, 0)),
            pl.BlockSpec((NBN, 3 * D1), lambda i: (i, 0)),
            pl.BlockSpec((D0, D0), lambda i: (0, 0)),
            pl.BlockSpec((D0, D0), lambda i: (0, 0)),
            pl.BlockSpec((D0, D0), lambda i: (0, 0)),
            pl.BlockSpec((D1, D1), lambda i: (0, 0)),
        ],
        out_specs=[
            pl.BlockSpec((NBN, D0), lambda i: (i, 0)),
            pl.BlockSpec((NBN, CPK), lambda i: (i, 0)),
        ],
        out_shape=[
            jax.ShapeDtypeStruct((NPAD, D0), jnp.bfloat16),
            jax.ShapeDtypeStruct((NPAD, CPK), jnp.int32),
        ],
    )(s_pad, v_pad, Wq, Wk, Wvs, Wvv)


def _post_kernel(s_ref, v_ref, msg_ref, wos_ref, wov_ref, wf1_ref, wf2_ref,
                 wg0_ref, wg1_ref, wg2_ref, so_ref, vo_ref):
    s = s_ref[...] + _dot(msg_ref[:, 0:D0], wos_ref[...])
    vparts = [v_ref[:, d * D1:(d + 1) * D1] +
              _dot(msg_ref[:, D0 + d * D1:D0 + (d + 1) * D1], wov_ref[...])
              for d in range(3)]
    s_n = _lnk(s)
    hidden = _dot(s_n, wf1_ref[...])
    s = s + _dot(hidden * jax.nn.sigmoid(hidden), wf2_ref[...])
    gate = jax.nn.sigmoid(_dot(s_n, wg0_ref[...]))
    so_ref[...] = s
    for d in range(3):
        vmid = _dot(vparts[d], wg1_ref[...]) * gate
        vo_ref[:, d * D1:(d + 1) * D1] = vparts[d] + _dot(vmid, wg2_ref[...])


def _post_layer(s_pad, v_pad, msg, Wos, Wov, Wf1, Wf2, Wg0, Wg1, Wg2):
    ws = (Wos, Wov, Wf1, Wf2, Wg0, Wg1, Wg2)
    specs_w = [pl.BlockSpec(w.shape, lambda i: (0, 0)) for w in ws]
    return pl.pallas_call(
        _post_kernel,
        grid=(NPAD // NBN,),
        in_specs=[
            pl.BlockSpec((NBN, D0), lambda i: (i, 0)),
            pl.BlockSpec((NBN, 3 * D1), lambda i: (i, 0)),
            pl.BlockSpec((NBN, CMSG), lambda i: (i, 0)),
        ] + specs_w,
        out_specs=[
            pl.BlockSpec((NBN, D0), lambda i: (i, 0)),
            pl.BlockSpec((NBN, 3 * D1), lambda i: (i, 0)),
        ],
        out_shape=[
            jax.ShapeDtypeStruct((NPAD, D0), jnp.float32),
            jax.ShapeDtypeStruct((NPAD, 3 * D1), jnp.float32),
        ],
    )(s_pad, v_pad, msg, *ws)


def _head_kernel(s_ref, b_ref, wh1_ref, wh2_ref, out_ref, acc):
    i = pl.program_id(0)

    @pl.when(i == 0)
    def _():
        acc[...] = jnp.zeros_like(acc)

    sf = _lnk(s_ref[...])
    hd = _dot(sf, wh1_ref[...])
    e = _dot(hd * jax.nn.sigmoid(hd), wh2_ref[...])
    cols = lax.broadcasted_iota(jnp.int32, (NBN, NG), 1)
    oh = (b_ref[...] == cols).astype(jnp.float32)
    acc[...] += lax.dot_general(oh, e, (((0,), (0,)), ((), ())),
                                preferred_element_type=jnp.float32)

    @pl.when(i == NPAD // NBN - 1)
    def _():
        out_ref[...] = acc[...] * (1.0 / AVG_NODES)


def _head(s_pad, batch_col, Wh1, Wh2p):
    return pl.pallas_call(
        _head_kernel,
        grid=(NPAD // NBN,),
        in_specs=[
            pl.BlockSpec((NBN, D0), lambda i: (i, 0)),
            pl.BlockSpec((NBN, 1), lambda i: (i, 0)),
            pl.BlockSpec((D0, D0), lambda i: (0, 0)),
            pl.BlockSpec((D0, 8), lambda i: (0, 0)),
        ],
        out_specs=pl.BlockSpec((NG, 8), lambda i: (0, 0)),
        scratch_shapes=[pltpu.VMEM((NG, 8), jnp.float32)],
        out_shape=jax.ShapeDtypeStruct((NG, 8), jnp.float32),
    )(s_pad, batch_col, Wh1, Wh2p)


def kernel(node_atom, node_tag, pos, edge_index, batch, atom_emb, tag_emb,
           Wd1, Wd2, Wd3s, Wd3v, Wq, Wk, Wvs, Wvv, We, Wos, Wov,
           Wf1, Wf2, Wg0, Wg1, Wg2, Wh1, Wh2):
    src = edge_index[0]
    dst = edge_index[1]
    n = pos.shape[0]

    perm = jnp.argsort(dst)
    srcp = src[perm].astype(jnp.int32)
    dstp = dst[perm].astype(jnp.int32)
    spread_idx = jnp.bitwise_and(jnp.arange(EPAD - E, dtype=jnp.int32), 8191)
    dstp_pad = jnp.concatenate(
        [dstp, jnp.full((EPAD - E,), NPAD - 1, jnp.int32)])
    srcp_pad = jnp.concatenate([srcp, spread_idx])
    dstp_clip = jnp.concatenate([dstp, spread_idx])
    p = jnp.searchsorted(dstp, jnp.arange(0, NPAD + 1, W, dtype=jnp.int32)
                         ).astype(jnp.int32)
    p = p.at[-1].set(E)

    s0 = atom_emb[node_atom] + tag_emb[node_tag]
    s0_pad = jnp.pad(s0, ((0, NPAD - n), (0, 0)))

    pos_pad = jnp.pad(pos, ((0, 0), (0, 13)))
    pos_src = pos_pad[srcp_pad]
    pos_dst = pos_pad[dstp_clip]
    We2 = jnp.concatenate([We[0], We[1]], axis=1)
    eb = _eb_pass(pos_src, pos_dst, We2)
    A = _degree_pass(p, dstp_pad, pos_src, pos_dst, Wd1, Wd2)
    s_pad, v_pad = _degproj(s0_pad, A, Wd3s, Wd3v)

    for l in range(L):
        q_pad, kvv_tab = _pre_layer(s_pad, v_pad, Wq[l], Wk[l], Wvs[l], Wvv[l])
        kvv_rows = _sc_gather(kvv_tab, srcp_pad)
        msg = _attn_pass(p, dstp_pad, kvv_rows, eb, q_pad, l)
        s_pad, v_pad = _post_layer(s_pad, v_pad, msg, Wos[l], Wov[l],
                                   Wf1[l], Wf2[l], Wg0[l], Wg1[l], Wg2[l])

    batch_col = jnp.pad(batch.astype(jnp.int32), (0, NPAD - n),
                        constant_values=-1).reshape(NPAD, 1)
    Wh2p = jnp.pad(Wh2, ((0, 0), (0, 7)))
    energy8 = _head(s_pad, batch_col, Wh1, Wh2p)
    return energy8[:, 0:1]


# speedup vs baseline: 5.3420x; 1.0153x over previous
import functools
import math

import jax
import jax.numpy as jnp
from jax import lax
from jax.experimental import pallas as pl
from jax.experimental.pallas import tpu as pltpu
from jax.experimental.pallas import tpu_sc as plsc

N = 10000
E = 160000
D0 = 256
D1 = 128
L = 2
H = 8
DH = 32
NB = 128
NG = 128
AVG_DEG = 23.395238876342773
AVG_NODES = 77.81317
MAX_R = 6.0

W = 128
NW = 80
NPAD = W * NW
B = 512
BA = 1024
EPAD = 163840
BE = 2048
NBN = 512
CMSG = D0 + D1 * 3
CKVV = D0 + CMSG
CPK = 512


def _lnk(x):
    m = x.mean(-1, keepdims=True)
    var = ((x - m) ** 2).mean(-1, keepdims=True)
    return (x - m) / jnp.sqrt(var + 1e-5)


def _dot(a, b):
    return lax.dot_general(a, b, (((1,), (0,)), ((), ())),
                           preferred_element_type=jnp.float32)


def _rbf_of(ps, pd):
    rel = ps[:, 0:3] - pd[:, 0:3]
    d2 = (rel * rel).sum(axis=1, keepdims=True) + 1e-12
    dist = jnp.sqrt(d2)
    step = MAX_R / (NB - 1)
    width = MAX_R / NB
    centers = lax.broadcasted_iota(jnp.int32, (1, NB), 1).astype(jnp.float32) * step
    t = (dist - centers) * (1.0 / width)
    return jnp.exp(-0.5 * t * t), rel, 1.0 / dist


NWK = 32
GCH = 80


def _sc_gather(table, idx):
    C = table.shape[1]
    dtype = table.dtype
    per_w = EPAD // NWK
    nch = per_w // GCH
    mesh = plsc.VectorSubcoreMesh(core_axis_name="c", subcore_axis_name="s")

    @functools.partial(
        pl.kernel, mesh=mesh,
        out_type=jax.ShapeDtypeStruct((EPAD, C), dtype),
        scratch_types=[
            pltpu.VMEM((GCH,), jnp.int32),
            pltpu.VMEM((GCH,), jnp.int32),
            pltpu.VMEM((GCH, C), dtype),
            pltpu.VMEM((GCH, C), dtype),
            pltpu.SemaphoreType.DMA,
            pltpu.SemaphoreType.DMA,
            pltpu.SemaphoreType.DMA,
            pltpu.SemaphoreType.DMA,
            pltpu.SemaphoreType.DMA,
            pltpu.SemaphoreType.DMA,
        ],
    )
    def k(table_hbm, idx_hbm, out_hbm, idx0, idx1, r0, r1,
          si0, si1, sg0, sg1, so0, so1):
        wid = lax.axis_index("s") * 2 + lax.axis_index("c")
        base = wid * per_w

        @pl.loop(0, nch // 2)
        def _(t):
            j0 = 2 * t
            j1 = j0 + 1
            i0 = pltpu.async_copy(
                idx_hbm.at[pl.ds(base + j0 * GCH, GCH)], idx0, si0)
            i1 = pltpu.async_copy(
                idx_hbm.at[pl.ds(base + j1 * GCH, GCH)], idx1, si1)
            i0.wait()
            c0 = pltpu.async_copy(table_hbm.at[idx0], r0, sg0)
            i1.wait()
            c1 = pltpu.async_copy(table_hbm.at[idx1], r1, sg1)
            c0.wait()
            w0 = pltpu.async_copy(r0, out_hbm.at[pl.ds(base + j0 * GCH, GCH)],
                                  so0)
            c1.wait()
            w1 = pltpu.async_copy(r1, out_hbm.at[pl.ds(base + j1 * GCH, GCH)],
                                  so1)
            w0.wait()
            w1.wait()

    return k(table, idx)


def _onehot(dst_col, base, nb=None):
    cols = lax.broadcasted_iota(jnp.int32, (nb or B, W), 1)
    return (dst_col - base == cols)


def _spread_s():
    rows = lax.broadcasted_iota(jnp.int32, (H, D0), 0)
    cols = lax.broadcasted_iota(jnp.int32, (H, D0), 1)
    return (cols // DH == rows).astype(jnp.float32)


def _spread_v():
    rows = lax.broadcasted_iota(jnp.int32, (H, D1 * 3), 0)
    cols = lax.broadcasted_iota(jnp.int32, (H, D1 * 3), 1)
    return ((cols % D1) // (D1 // H) == rows).astype(jnp.float32)


def _spread_sv():
    return jnp.concatenate([_spread_s(), _spread_v()], axis=1)


def _headsum():
    rows = lax.broadcasted_iota(jnp.int32, (H * DH, H), 0)
    cols = lax.broadcasted_iota(jnp.int32, (H * DH, H), 1)
    return (rows // DH == cols).astype(jnp.bfloat16)


def _degree_kernel(p_ref, dst_ref, ps_ref, pd_ref, Wd1_ref, Wd2_ref, We2_ref,
                   A_ref, eb_ref,
                   dst_v0, ps_v0, pd_v0, dst_v1, ps_v1, pd_v1, acc, eb_v,
                   s10, s20, s30, s11, s21, s31, s4):
    w = pl.program_id(0)
    start = p_ref[w]
    end = jnp.where(w == NW - 1, EPAD, p_ref[w + 1])
    base = w * W
    k0 = start // B
    nch = (end + B - 1) // B - k0
    acc[...] = jnp.zeros_like(acc)

    bufs = ((dst_v0, ps_v0, pd_v0, s10, s20, s30),
            (dst_v1, ps_v1, pd_v1, s11, s21, s31))

    def start_chunk(b, j):
        dv, pv, qv, sa, sb, sc = bufs[b]
        off = (k0 + j) * B
        pltpu.make_async_copy(dst_ref.at[pl.ds(off, B)], dv, sa).start()
        pltpu.make_async_copy(ps_ref.at[pl.ds(off, B), :], pv, sb).start()
        pltpu.make_async_copy(pd_ref.at[pl.ds(off, B), :], qv, sc).start()

    def compute_chunk(b, j):
        dv, pv, qv, sa, sb, sc = bufs[b]
        off = (k0 + j) * B
        pltpu.make_async_copy(dst_ref.at[pl.ds(off, B)], dv, sa).wait()
        pltpu.make_async_copy(ps_ref.at[pl.ds(off, B), :], pv, sb).wait()
        pltpu.make_async_copy(pd_ref.at[pl.ds(off, B), :], qv, sc).wait()
        rbf, rel, inv = _rbf_of(pv[...], qv[...])
        eb_v[...] = rbf @ We2_ref[...]
        wcp = pltpu.make_async_copy(eb_v, eb_ref.at[pl.ds(off, B), :], s4)
        wcp.start()
        h = rbf @ Wd1_ref[...]
        h = h * jax.nn.sigmoid(h)
        h = h @ Wd2_ref[...]
        h = h * jax.nn.sigmoid(h)
        sh = rel * inv
        hh = jnp.concatenate(
            [h, h * sh[:, 0:1], h * sh[:, 1:2], h * sh[:, 2:3]],
            axis=1).astype(jnp.bfloat16)
        oh = _onehot(dv[...].reshape(B, 1), base).astype(jnp.bfloat16)
        acc[...] += lax.dot_general(oh, hh, (((0,), (0,)), ((), ())),
                                    preferred_element_type=jnp.float32)
        wcp.wait()

    @pl.when(nch > 0)
    def _():
        start_chunk(0, 0)

    def body2(t, _):
        j0 = 2 * t
        j1 = j0 + 1

        @pl.when(j1 < nch)
        def _():
            start_chunk(1, j1)

        compute_chunk(0, j0)

        @pl.when(j1 + 1 < nch)
        def _():
            start_chunk(0, j1 + 1)

        @pl.when(j1 < nch)
        def _():
            compute_chunk(1, j1)

        return 0

    lax.fori_loop(0, (nch + 1) // 2, body2, 0)
    A_ref[...] = acc[...]


def _degree_pass(p, dstp, pos_src, pos_dst, Wd1, Wd2, We2):
    grid_spec = pltpu.PrefetchScalarGridSpec(
        num_scalar_prefetch=1,
        grid=(NW,),
        in_specs=[
            pl.BlockSpec(memory_space=pltpu.MemorySpace.HBM),
            pl.BlockSpec(memory_space=pltpu.MemorySpace.HBM),
            pl.BlockSpec(memory_space=pltpu.MemorySpace.HBM),
            pl.BlockSpec((NB, 64), lambda w, p: (0, 0)),
            pl.BlockSpec((64, 64), lambda w, p: (0, 0)),
            pl.BlockSpec((NB, 2 * H), lambda w, p: (0, 0)),
        ],
        out_specs=[
            pl.BlockSpec((W, 256), lambda w, p: (w, 0)),
            pl.BlockSpec(memory_space=pltpu.MemorySpace.HBM),
        ],
        scratch_shapes=[
            pltpu.VMEM((B,), jnp.int32),
            pltpu.VMEM((B, 16), jnp.float32),
            pltpu.VMEM((B, 16), jnp.float32),
            pltpu.VMEM((B,), jnp.int32),
            pltpu.VMEM((B, 16), jnp.float32),
            pltpu.VMEM((B, 16), jnp.float32),
            pltpu.VMEM((W, 256), jnp.float32),
            pltpu.VMEM((B, 2 * H), jnp.float32),
            pltpu.SemaphoreType.DMA,
            pltpu.SemaphoreType.DMA,
            pltpu.SemaphoreType.DMA,
            pltpu.SemaphoreType.DMA,
            pltpu.SemaphoreType.DMA,
            pltpu.SemaphoreType.DMA,
            pltpu.SemaphoreType.DMA,
        ],
    )
    return pl.pallas_call(
        _degree_kernel,
        grid_spec=grid_spec,
        out_shape=[
            jax.ShapeDtypeStruct((NPAD, 256), jnp.float32),
            jax.ShapeDtypeStruct((EPAD, 2 * H), jnp.float32),
        ],
    )(p, dstp, pos_src, pos_dst, Wd1, Wd2, We2)


def _make_attn_kernel(l):
    def _attn_kernel(p_ref, dst_ref, kvv_ref, eb_ref, q_ref, out_ref,
                     dst_v0, kvv_v0, eb_v0, dst_v1, kvv_v1, eb_v1,
                     acc, s10, s20, s30, s11, s21, s31):
        w = pl.program_id(0)
        start = p_ref[w]
        end = p_ref[w + 1]
        base = w * W
        k0 = start // BA
        nch = (end + BA - 1) // BA - k0
        acc[...] = jnp.zeros_like(acc)

        bufs = ((dst_v0, kvv_v0, eb_v0, s10, s20, s30),
                (dst_v1, kvv_v1, eb_v1, s11, s21, s31))

        def start_chunk(b, j):
            dv, kv, ev, sa, sb, sc = bufs[b]
            off = (k0 + j) * BA
            pltpu.make_async_copy(dst_ref.at[pl.ds(off, BA)], dv, sa).start()
            pltpu.make_async_copy(kvv_ref.at[pl.ds(off, BA), :], kv, sb).start()
            pltpu.make_async_copy(eb_ref.at[pl.ds(off, BA), :], ev, sc).start()

        def compute_chunk(b, j):
            dv, kv, ev, sa, sb, sc = bufs[b]
            off = (k0 + j) * BA
            pltpu.make_async_copy(dst_ref.at[pl.ds(off, BA)], dv, sa).wait()
            pltpu.make_async_copy(kvv_ref.at[pl.ds(off, BA), :], kv, sb).wait()
            pltpu.make_async_copy(eb_ref.at[pl.ds(off, BA), :], ev, sc).wait()
            val = kv[:, 0:448]
            even = lax.bitcast_convert_type(lax.shift_left(val, 16), jnp.float32)
            odd = lax.bitcast_convert_type(
                jnp.bitwise_and(val, jnp.int32(-65536)), jnp.float32)
            kvv = jnp.concatenate([even, odd], axis=1)
            oh = _onehot(dv[...].reshape(BA, 1), base, BA).astype(jnp.bfloat16)
            q_e = lax.dot_general(oh, q_ref[...], (((1,), (0,)), ((), ())),
                                  preferred_element_type=jnp.float32)
            prod = (q_e * kvv[:, 0:D0]).astype(jnp.bfloat16)
            alpha = lax.dot_general(prod, _headsum(), (((1,), (0,)), ((), ())),
                                    preferred_element_type=jnp.float32)
            ex = jnp.exp(jnp.clip(alpha + ev[:, l * H:(l + 1) * H], -60.0, 60.0))
            w_sv = lax.dot_general(ex, _spread_sv(), (((1,), (0,)), ((), ())),
                                   preferred_element_type=jnp.float32)
            payload = jnp.concatenate(
                [kvv[:, D0:CKVV] * w_sv, ex], axis=1).astype(jnp.bfloat16)
            acc[...] += lax.dot_general(oh, payload, (((0,), (0,)), ((), ())),
                                        preferred_element_type=jnp.float32)

        @pl.when(nch > 0)
        def _():
            start_chunk(0, 0)

        def body2(t, _):
            j0 = 2 * t
            j1 = j0 + 1

            @pl.when(j1 < nch)
            def _():
                start_chunk(1, j1)

            compute_chunk(0, j0)

            @pl.when(j1 + 1 < nch)
            def _():
                start_chunk(0, j1 + 1)

            @pl.when(j1 < nch)
            def _():
                compute_chunk(1, j1)

            return 0

        lax.fori_loop(0, (nch + 1) // 2, body2, 0)
        dfull = lax.dot_general(acc[:, CMSG:CMSG + H], _spread_sv(),
                                (((1,), (0,)), ((), ())),
                                preferred_element_type=jnp.float32)
        out_ref[...] = jnp.where(dfull > 0.0, acc[:, 0:CMSG] / dfull, 0.0)

    return _attn_kernel


def _attn_pass(p, dstp, kvv_rows, eb, q_pad, l):
    grid_spec = pltpu.PrefetchScalarGridSpec(
        num_scalar_prefetch=1,
        grid=(NW,),
        in_specs=[
            pl.BlockSpec(memory_space=pltpu.MemorySpace.HBM),
            pl.BlockSpec(memory_space=pltpu.MemorySpace.HBM),
            pl.BlockSpec(memory_space=pltpu.MemorySpace.HBM),
            pl.BlockSpec((W, D0), lambda w, p: (w, 0)),
        ],
        out_specs=pl.BlockSpec((W, CMSG), lambda w, p: (w, 0)),
        scratch_shapes=[
            pltpu.VMEM((BA,), jnp.int32),
            pltpu.VMEM((BA, CPK), jnp.int32),
            pltpu.VMEM((BA, 2 * H), jnp.float32),
            pltpu.VMEM((BA,), jnp.int32),
            pltpu.VMEM((BA, CPK), jnp.int32),
            pltpu.VMEM((BA, 2 * H), jnp.float32),
            pltpu.VMEM((W, CMSG + H), jnp.float32),
            pltpu.SemaphoreType.DMA,
            pltpu.SemaphoreType.DMA,
            pltpu.SemaphoreType.DMA,
            pltpu.SemaphoreType.DMA,
            pltpu.SemaphoreType.DMA,
            pltpu.SemaphoreType.DMA,
        ],
    )
    return pl.pallas_call(
        _make_attn_kernel(l),
        grid_spec=grid_spec,
        out_shape=jax.ShapeDtypeStruct((NPAD, CMSG), jnp.float32),
    )(p, dstp, kvv_rows, eb, q_pad)


def _degproj_kernel(s0_ref, A_ref, Wd3s_ref, Wd3v_ref, s_ref, v_ref):
    c = 1.0 / math.sqrt(AVG_DEG)
    s_ref[...] = s0_ref[...] + _dot(A_ref[:, 0:64], Wd3s_ref[...]) * c
    for d in range(3):
        v_ref[:, d * D1:(d + 1) * D1] = _dot(
            A_ref[:, 64 + 64 * d:128 + 64 * d], Wd3v_ref[...]) * c


def _degproj(s0_pad, A, Wd3s, Wd3v):
    return pl.pallas_call(
        _degproj_kernel,
        grid=(NPAD // NBN,),
        in_specs=[
            pl.BlockSpec((NBN, D0), lambda i: (i, 0)),
            pl.BlockSpec((NBN, 256), lambda i: (i, 0)),
            pl.BlockSpec((64, D0), lambda i: (0, 0)),
            pl.BlockSpec((64, D1), lambda i: (0, 0)),
        ],
        out_specs=[
            pl.BlockSpec((NBN, D0), lambda i: (i, 0)),
            pl.BlockSpec((NBN, 3 * D1), lambda i: (i, 0)),
        ],
        out_shape=[
            jax.ShapeDtypeStruct((NPAD, D0), jnp.float32),
            jax.ShapeDtypeStruct((NPAD, 3 * D1), jnp.float32),
        ],
    )(s0_pad, A, Wd3s, Wd3v)


def _bf16_bits(x):
    r = x.astype(jnp.bfloat16).astype(jnp.float32)
    return lax.bitcast_convert_type(r, jnp.int32)


def _pre_kernel(s_ref, v_ref, wq_ref, wk_ref, wvs_ref, wvv_ref, q_ref, kvv_ref):
    s_in = _lnk(s_ref[...])
    q_ref[...] = (_dot(s_in, wq_ref[...]) * (1.0 / math.sqrt(DH))
                  ).astype(jnp.bfloat16)
    kres = _dot(s_in, wk_ref[...])
    vsres = _dot(s_in, wvs_ref[...])
    vvs = [_dot(v_ref[:, d * D1:(d + 1) * D1], wvv_ref[...]) for d in range(3)]
    full_l = jnp.concatenate([kres, vsres[:, 0:448 - D0]], axis=1)
    full_r = jnp.concatenate([vsres[:, 448 - D0:], vvs[0], vvs[1], vvs[2],
                              jnp.zeros((NBN, CPK - 448), jnp.float32)], axis=1)
    lb = lax.shift_right_logical(_bf16_bits(full_l), 16)
    rb = jnp.bitwise_and(_bf16_bits(full_r), jnp.int32(-65536))
    kvv_ref[...] = jnp.bitwise_or(
        jnp.concatenate([lb, jnp.zeros((NBN, CPK - 448), jnp.int32)], axis=1),
        rb)


def _pre_layer(s_pad, v_pad, Wq, Wk, Wvs, Wvv):
    return pl.pallas_call(
        _pre_kernel,
        grid=(NPAD // NBN,),
        in_specs=[
            pl.BlockSpec((NBN, D0), lambda i: (i, 0)),
            pl.BlockSpec((NBN, 3 * D1), lambda i: (i, 0)),
            pl.BlockSpec((D0, D0), lambda i: (0, 0)),
            pl.BlockSpec((D0, D0), lambda i: (0, 0)),
            pl.BlockSpec((D0, D0), lambda i: (0, 0)),
            pl.BlockSpec((D1, D1), lambda i: (0, 0)),
        ],
        out_specs=[
            pl.BlockSpec((NBN, D0), lambda i: (i, 0)),
            pl.BlockSpec((NBN, CPK), lambda i: (i, 0)),
        ],
        out_shape=[
            jax.ShapeDtypeStruct((NPAD, D0), jnp.bfloat16),
            jax.ShapeDtypeStruct((NPAD, CPK), jnp.int32),
        ],
    )(s_pad, v_pad, Wq, Wk, Wvs, Wvv)


def _post_kernel(s_ref, v_ref, msg_ref, wos_ref, wov_ref, wf1_ref, wf2_ref,
                 wg0_ref, wg1_ref, wg2_ref, so_ref, vo_ref):
    s = s_ref[...] + _dot(msg_ref[:, 0:D0], wos_ref[...])
    vparts = [v_ref[:, d * D1:(d + 1) * D1] +
              _dot(msg_ref[:, D0 + d * D1:D0 + (d + 1) * D1], wov_ref[...])
              for d in range(3)]
    s_n = _lnk(s)
    hidden = _dot(s_n, wf1_ref[...])
    s = s + _dot(hidden * jax.nn.sigmoid(hidden), wf2_ref[...])
    gate = jax.nn.sigmoid(_dot(s_n, wg0_ref[...]))
    so_ref[...] = s
    for d in range(3):
        vmid = _dot(vparts[d], wg1_ref[...]) * gate
        vo_ref[:, d * D1:(d + 1) * D1] = vparts[d] + _dot(vmid, wg2_ref[...])


def _post_layer(s_pad, v_pad, msg, Wos, Wov, Wf1, Wf2, Wg0, Wg1, Wg2):
    ws = (Wos, Wov, Wf1, Wf2, Wg0, Wg1, Wg2)
    specs_w = [pl.BlockSpec(w.shape, lambda i: (0, 0)) for w in ws]
    return pl.pallas_call(
        _post_kernel,
        grid=(NPAD // NBN,),
        in_specs=[
            pl.BlockSpec((NBN, D0), lambda i: (i, 0)),
            pl.BlockSpec((NBN, 3 * D1), lambda i: (i, 0)),
            pl.BlockSpec((NBN, CMSG), lambda i: (i, 0)),
        ] + specs_w,
        out_specs=[
            pl.BlockSpec((NBN, D0), lambda i: (i, 0)),
            pl.BlockSpec((NBN, 3 * D1), lambda i: (i, 0)),
        ],
        out_shape=[
            jax.ShapeDtypeStruct((NPAD, D0), jnp.float32),
            jax.ShapeDtypeStruct((NPAD, 3 * D1), jnp.float32),
        ],
    )(s_pad, v_pad, msg, *ws)


def _head_kernel(s_ref, b_ref, wh1_ref, wh2_ref, out_ref, acc):
    i = pl.program_id(0)

    @pl.when(i == 0)
    def _():
        acc[...] = jnp.zeros_like(acc)

    sf = _lnk(s_ref[...])
    hd = _dot(sf, wh1_ref[...])
    e = _dot(hd * jax.nn.sigmoid(hd), wh2_ref[...])
    cols = lax.broadcasted_iota(jnp.int32, (NBN, NG), 1)
    oh = (b_ref[...] == cols).astype(jnp.float32)
    acc[...] += lax.dot_general(oh, e, (((0,), (0,)), ((), ())),
                                preferred_element_type=jnp.float32)

    @pl.when(i == NPAD // NBN - 1)
    def _():
        out_ref[...] = acc[...] * (1.0 / AVG_NODES)


def _head(s_pad, batch_col, Wh1, Wh2p):
    return pl.pallas_call(
        _head_kernel,
        grid=(NPAD // NBN,),
        in_specs=[
            pl.BlockSpec((NBN, D0), lambda i: (i, 0)),
            pl.BlockSpec((NBN, 1), lambda i: (i, 0)),
            pl.BlockSpec((D0, D0), lambda i: (0, 0)),
            pl.BlockSpec((D0, 8), lambda i: (0, 0)),
        ],
        out_specs=pl.BlockSpec((NG, 8), lambda i: (0, 0)),
        scratch_shapes=[pltpu.VMEM((NG, 8), jnp.float32)],
        out_shape=jax.ShapeDtypeStruct((NG, 8), jnp.float32),
    )(s_pad, batch_col, Wh1, Wh2p)


def kernel(node_atom, node_tag, pos, edge_index, batch, atom_emb, tag_emb,
           Wd1, Wd2, Wd3s, Wd3v, Wq, Wk, Wvs, Wvv, We, Wos, Wov,
           Wf1, Wf2, Wg0, Wg1, Wg2, Wh1, Wh2):
    src = edge_index[0]
    dst = edge_index[1]
    n = pos.shape[0]

    perm = jnp.argsort(dst)
    srcp = src[perm].astype(jnp.int32)
    dstp = dst[perm].astype(jnp.int32)
    spread_idx = jnp.bitwise_and(jnp.arange(EPAD - E, dtype=jnp.int32), 8191)
    dstp_pad = jnp.concatenate(
        [dstp, jnp.full((EPAD - E,), NPAD - 1, jnp.int32)])
    srcp_pad = jnp.concatenate([srcp, spread_idx])
    dstp_clip = jnp.concatenate([dstp, spread_idx])
    p = jnp.searchsorted(dstp, jnp.arange(0, NPAD + 1, W, dtype=jnp.int32)
                         ).astype(jnp.int32)
    p = p.at[-1].set(E)

    s0 = atom_emb[node_atom] + tag_emb[node_tag]
    s0_pad = jnp.pad(s0, ((0, NPAD - n), (0, 0)))

    pos_pad = jnp.pad(pos, ((0, 0), (0, 13)))
    pos_src = pos_pad[srcp_pad]
    pos_dst = pos_pad[dstp_clip]
    We2 = jnp.concatenate([We[0], We[1]], axis=1)
    A, eb = _degree_pass(p, dstp_pad, pos_src, pos_dst, Wd1, Wd2, We2)
    s_pad, v_pad = _degproj(s0_pad, A, Wd3s, Wd3v)

    for l in range(L):
        q_pad, kvv_tab = _pre_layer(s_pad, v_pad, Wq[l], Wk[l], Wvs[l], Wvv[l])
        kvv_rows = _sc_gather(kvv_tab, srcp_pad)
        msg = _attn_pass(p, dstp_pad, kvv_rows, eb, q_pad, l)
        s_pad, v_pad = _post_layer(s_pad, v_pad, msg, Wos[l], Wov[l],
                                   Wf1[l], Wf2[l], Wg0[l], Wg1[l], Wg2[l])

    batch_col = jnp.pad(batch.astype(jnp.int32), (0, NPAD - n),
                        constant_values=-1).reshape(NPAD, 1)
    Wh2p = jnp.pad(Wh2, ((0, 0), (0, 7)))
    energy8 = _head(s_pad, batch_col, Wh1, Wh2p)
    return energy8[:, 0:1]
